# Initial kernel scaffold; baseline (speedup 1.0000x reference)
import jax
import jax.numpy as jnp
from jax.experimental import pallas as pl


def kernel(x, edge_index, edge_attr, batch, W0, b0, We1, be1, We2, be2, root, bias, gWih, gWhh, gbih, gbhh, sWih, sWhh, sbih, sbhh, W1, b1, W2, b2):
    raise NotImplementedError("write your pallas kernel here")


# speedup vs baseline: 1.0002x; 1.0002x over previous
import functools

import jax
import jax.numpy as jnp
from jax import lax
from jax.experimental import pallas as pl
from jax.experimental.pallas import tpu as pltpu
from jax.experimental.pallas import tpu_sc as plsc

_NC, _NS = 2, 16
_NW = _NC * _NS
_CH = 40


def _worker_id():
    return lax.axis_index("s") * _NC + lax.axis_index("c")


def _sc_gather(nodes, idx3d, H):
    _, NCH, CH = idx3d.shape
    E = _NW * NCH * CH
    mesh = plsc.VectorSubcoreMesh(core_axis_name="c", subcore_axis_name="s")

    @functools.partial(
        pl.kernel,
        out_type=jax.ShapeDtypeStruct((E, H), jnp.float32),
        mesh=mesh,
        compiler_params=pltpu.CompilerParams(use_tc_tiling_on_sc=False),
        scratch_types=[
            pltpu.VMEM((NCH, CH), jnp.int32),
            pltpu.VMEM((CH, H), jnp.float32),
            pltpu.SemaphoreType.DMA,
        ],
    )
    def k(nodes_hbm, idx_hbm, out_hbm, idx_v, rows_v, sem):
        wid = _worker_id()
        pltpu.sync_copy(idx_hbm.at[wid], idx_v)

        def body(j, carry):
            pltpu.async_copy(nodes_hbm.at[idx_v.at[j]], rows_v, sem).wait()
            pltpu.sync_copy(rows_v, out_hbm.at[pl.ds((wid * NCH + j) * CH, CH)])
            return carry

        lax.fori_loop(0, NCH, body, 0)

    return k(nodes, idx3d)


def _sc_scatter_add(rows, idx3d, zeros_nh, H):
    _, NCH, CH = idx3d.shape
    NPAD = zeros_nh.shape[0]
    n_per_s = NPAD // _NS
    mesh = plsc.VectorSubcoreMesh(core_axis_name="c", subcore_axis_name="s")

    @functools.partial(
        pl.kernel,
        out_type=jax.ShapeDtypeStruct((_NC, NPAD, H), jnp.float32),
        mesh=mesh,
        compiler_params=pltpu.CompilerParams(use_tc_tiling_on_sc=False),
        scratch_types=[
            pltpu.VMEM((NCH, CH), jnp.int32),
            pltpu.VMEM((CH, H), jnp.float32),
            pltpu.VMEM_SHARED((NPAD, H), jnp.float32),
        ],
    )
    def k(rows_hbm, idx_hbm, zeros_hbm, out_hbm, idx_v, rows_v, acc):
        c = lax.axis_index("c")
        s = lax.axis_index("s")
        wid = s * _NC + c
        pltpu.sync_copy(zeros_hbm.at[pl.ds(s * n_per_s, n_per_s)],
                        acc.at[pl.ds(s * n_per_s, n_per_s)])
        pltpu.sync_copy(idx_hbm.at[wid], idx_v)
        plsc.subcore_barrier()

        def body(j, carry):
            pltpu.sync_copy(rows_hbm.at[pl.ds((wid * NCH + j) * CH, CH)],
                            rows_v)
            pltpu.sync_copy(rows_v, acc.at[idx_v.at[j]], add=True)
            return carry

        lax.fori_loop(0, NCH, body, 0)
        plsc.subcore_barrier()
        pltpu.sync_copy(acc.at[pl.ds(s * n_per_s, n_per_s)],
                        out_hbm.at[c, pl.ds(s * n_per_s, n_per_s)])

    return k(rows, idx3d, zeros_nh)


def _sc_degree_count(idx3d, ones_rows, zeros_nh, H):
    _, NCH, CH = idx3d.shape
    NPAD = zeros_nh.shape[0]
    n_per_s = NPAD // _NS
    mesh = plsc.VectorSubcoreMesh(core_axis_name="c", subcore_axis_name="s")

    @functools.partial(
        pl.kernel,
        out_type=jax.ShapeDtypeStruct((_NC, NPAD, H), jnp.float32),
        mesh=mesh,
        compiler_params=pltpu.CompilerParams(use_tc_tiling_on_sc=False),
        scratch_types=[
            pltpu.VMEM((NCH, CH), jnp.int32),
            pltpu.VMEM((CH, H), jnp.float32),
            pltpu.VMEM_SHARED((NPAD, H), jnp.float32),
        ],
    )
    def k(idx_hbm, ones_hbm, zeros_hbm, out_hbm, idx_v, rows_v, acc):
        c = lax.axis_index("c")
        s = lax.axis_index("s")
        wid = s * _NC + c
        pltpu.sync_copy(zeros_hbm.at[pl.ds(s * n_per_s, n_per_s)],
                        acc.at[pl.ds(s * n_per_s, n_per_s)])
        pltpu.sync_copy(idx_hbm.at[wid], idx_v)
        pltpu.sync_copy(ones_hbm, rows_v)
        plsc.subcore_barrier()

        def body(j, carry):
            pltpu.sync_copy(rows_v, acc.at[idx_v.at[j]], add=True)
            return carry

        lax.fori_loop(0, NCH, body, 0)
        plsc.subcore_barrier()
        pltpu.sync_copy(acc.at[pl.ds(s * n_per_s, n_per_s)],
                        out_hbm.at[c, pl.ds(s * n_per_s, n_per_s)])

    return k(idx3d, ones_rows, zeros_nh)


def _tc_input_proj(x, W0T, b0r):
    N = x.shape[0]
    H = W0T.shape[1]

    def body(x_ref, w_ref, b_ref, o_ref):
        o_ref[...] = jax.nn.relu(
            jnp.dot(x_ref[...], w_ref[...], preferred_element_type=jnp.float32)
            + b_ref[...])

    return pl.pallas_call(
        body, out_shape=jax.ShapeDtypeStruct((N, H), jnp.float32))(x, W0T, b0r)


def _tc_edge_mlp(edge_attr, We1T, be1r):
    E = edge_attr.shape[0]
    K = We1T.shape[1]
    EB = 8000
    grid = (E // EB,)

    def body(ea_ref, w_ref, b_ref, o_ref):
        f = jax.nn.relu(
            jnp.dot(ea_ref[...], w_ref[...], preferred_element_type=jnp.float32)
            + b_ref[...])
        o_ref[...] = f.astype(jnp.bfloat16)

    return pl.pallas_call(
        body,
        grid=grid,
        in_specs=[
            pl.BlockSpec((EB, edge_attr.shape[1]), lambda i: (i, 0)),
            pl.BlockSpec(We1T.shape, lambda i: (0, 0)),
            pl.BlockSpec(be1r.shape, lambda i: (0, 0)),
        ],
        out_specs=pl.BlockSpec((EB, K), lambda i: (i, 0)),
        out_shape=jax.ShapeDtypeStruct((E, K), jnp.bfloat16),
    )(edge_attr, We1T, be1r)


def _tc_messages(f, xj, We2T, be2m, H):
    E, K = f.shape
    EB = 640
    grid = (E // EB,)

    def body(f_ref, xj_ref, w_ref, bm_ref, o_ref):
        we = jnp.dot(f_ref[...], w_ref[...], preferred_element_type=jnp.float32)
        xj = xj_ref[...]
        msg = jnp.dot(xj, bm_ref[...], preferred_element_type=jnp.float32)
        for i in range(H):
            msg = msg + xj[:, i:i + 1] * we[:, i * H:(i + 1) * H]
        o_ref[...] = msg

    return pl.pallas_call(
        body,
        grid=grid,
        in_specs=[
            pl.BlockSpec((EB, K), lambda i: (i, 0)),
            pl.BlockSpec((EB, H), lambda i: (i, 0)),
            pl.BlockSpec(We2T.shape, lambda i: (0, 0)),
            pl.BlockSpec(be2m.shape, lambda i: (0, 0)),
        ],
        out_specs=pl.BlockSpec((EB, H), lambda i: (i, 0)),
        out_shape=jax.ShapeDtypeStruct((E, H), jnp.float32),
    )(f, xj, We2T, be2m)


def _tc_gru(parts, cparts, prev, rootm, biasr, gWihT, gWhhT, gbihr, gbhhr, H):
    N = prev.shape[0]
    NB = 1000
    grid = (N // NB,)

    def body(p_ref, c_ref, h_ref, root_ref, bias_ref, wih_ref, whh_ref,
             bih_ref, bhh_ref, o_ref):
        cnt = jnp.clip(c_ref[0, :, 0:1] + c_ref[1, :, 0:1], 1.0, None)
        agg = (p_ref[0] + p_ref[1]) / cnt
        h = h_ref[...]
        m = jax.nn.relu(
            agg + jnp.dot(h, root_ref[...], preferred_element_type=jnp.float32)
            + bias_ref[...])
        gi = jnp.dot(m, wih_ref[...], preferred_element_type=jnp.float32) + bih_ref[...]
        gh = jnp.dot(h, whh_ref[...], preferred_element_type=jnp.float32) + bhh_ref[...]
        r = jax.nn.sigmoid(gi[:, 0:H] + gh[:, 0:H])
        z = jax.nn.sigmoid(gi[:, H:2 * H] + gh[:, H:2 * H])
        n = jnp.tanh(gi[:, 2 * H:3 * H] + r * gh[:, 2 * H:3 * H])
        o_ref[...] = (1.0 - z) * n + z * h

    return pl.pallas_call(
        body,
        grid=grid,
        in_specs=[
            pl.BlockSpec((2, NB, H), lambda i: (0, i, 0)),
            pl.BlockSpec((2, NB, H), lambda i: (0, i, 0)),
            pl.BlockSpec((NB, H), lambda i: (i, 0)),
            pl.BlockSpec(rootm.shape, lambda i: (0, 0)),
            pl.BlockSpec(biasr.shape, lambda i: (0, 0)),
            pl.BlockSpec(gWihT.shape, lambda i: (0, 0)),
            pl.BlockSpec(gWhhT.shape, lambda i: (0, 0)),
            pl.BlockSpec(gbihr.shape, lambda i: (0, 0)),
            pl.BlockSpec(gbhhr.shape, lambda i: (0, 0)),
        ],
        out_specs=pl.BlockSpec((NB, H), lambda i: (i, 0)),
        out_shape=jax.ShapeDtypeStruct((N, H), jnp.float32),
    )(parts, cparts, prev, rootm, biasr, gWihT, gWhhT, gbihr, gbhhr)


def _tc_set2set(out, batch_col, sWihT, sWhhT, sbihr, sbhhr, W1T, b1r, W2T, b2r,
                B, H, PS):
    def body(out_ref, b_ref, wih_ref, whh_ref, bih_ref, bhh_ref,
             w1_ref, b1_ref, w2_ref, b2_ref, o_ref):
        nodes = out_ref[...]
        bcol = b_ref[...]
        onehot = (bcol == lax.broadcasted_iota(jnp.int32, (1, B), 1)
                  ).astype(jnp.float32)
        q_star = jnp.zeros((B, 2 * H), jnp.float32)
        hl = jnp.zeros((B, H), jnp.float32)
        cl = jnp.zeros((B, H), jnp.float32)
        for _ in range(PS):
            g = (jnp.dot(q_star, wih_ref[...], preferred_element_type=jnp.float32)
                 + bih_ref[...]
                 + jnp.dot(hl, whh_ref[...], preferred_element_type=jnp.float32)
                 + bhh_ref[...])
            cl = (jax.nn.sigmoid(g[:, H:2 * H]) * cl
                  + jax.nn.sigmoid(g[:, 0:H]) * jnp.tanh(g[:, 2 * H:3 * H]))
            hl = jax.nn.sigmoid(g[:, 3 * H:4 * H]) * jnp.tanh(cl)
            q_b = jnp.dot(onehot, hl, preferred_element_type=jnp.float32)
            e = jnp.sum(nodes * q_b, axis=1, keepdims=True)
            emax = jnp.max(jnp.where(onehot > 0.0, e, -jnp.inf), axis=0,
                           keepdims=True)
            emax_b = jnp.sum(onehot * emax, axis=1, keepdims=True)
            a = jnp.exp(e - emax_b)
            asum = jnp.sum(onehot * a, axis=0, keepdims=True)
            asum_b = jnp.sum(onehot * asum, axis=1, keepdims=True)
            a = a / (asum_b + 1e-16)
            r_ = lax.dot_general(onehot * a, nodes, (((0,), (0,)), ((), ())),
                                 preferred_element_type=jnp.float32)
            q_star = jnp.concatenate([hl, r_], axis=1)
        o = jax.nn.relu(
            jnp.dot(q_star, w1_ref[...], preferred_element_type=jnp.float32)
            + b1_ref[...])
        o_ref[...] = (jnp.dot(o, w2_ref[...], preferred_element_type=jnp.float32)
                      + b2_ref[...])

    return pl.pallas_call(
        body, out_shape=jax.ShapeDtypeStruct((B, 1), jnp.float32),
    )(out, batch_col, sWihT, sWhhT, sbihr, sbhhr, W1T, b1r, W2T, b2r)


def kernel(x, edge_index, edge_attr, batch, W0, b0, We1, be1, We2, be2, root,
           bias, gWih, gWhh, gbih, gbhh, sWih, sWhh, sbih, sbhh, W1, b1, W2,
           b2):
    N = x.shape[0]
    E = edge_index.shape[1]
    H = root.shape[0]
    B = 64
    L, PS = 3, 3
    NCH = E // (_NW * _CH)
    NPAD = ((N + 8 * _NS - 1) // (8 * _NS)) * (8 * _NS)

    src3d = edge_index[0].reshape(_NW, NCH, _CH)
    dst3d = edge_index[1].reshape(_NW, NCH, _CH)
    zeros_nh = jnp.zeros((NPAD, H), jnp.float32)
    ones_rows = jnp.ones((_CH, H), jnp.float32)
    batch_col = batch.reshape(N, 1)

    out = _tc_input_proj(x, W0.T, b0.reshape(1, H))
    f = _tc_edge_mlp(edge_attr, We1.T, be1.reshape(1, -1))
    cparts = _sc_degree_count(dst3d, ones_rows, zeros_nh, H)

    We2T = We2.T.astype(jnp.bfloat16)
    be2m = be2.reshape(H, H)
    gWihT, gWhhT = gWih.T, gWhh.T
    gbihr, gbhhr = gbih.reshape(1, -1), gbhh.reshape(1, -1)
    biasr = bias.reshape(1, H)

    for _ in range(L):
        xj = _sc_gather(out, src3d, H)
        msg = _tc_messages(f, xj, We2T, be2m, H)
        parts = _sc_scatter_add(msg, dst3d, zeros_nh, H)
        out = _tc_gru(parts, cparts, out, root, biasr, gWihT, gWhhT,
                      gbihr, gbhhr, H)

    o = _tc_set2set(out, batch_col, sWih.T, sWhh.T, sbih.reshape(1, -1),
                    sbhh.reshape(1, -1), W1.T, b1.reshape(1, H), W2.T,
                    b2.reshape(1, 1), B, H, PS)
    return o.reshape(-1)


# speedup vs baseline: 2.6019x; 2.6014x over previous
import functools

import jax
import jax.numpy as jnp
from jax import lax
from jax.experimental import pallas as pl
from jax.experimental.pallas import tpu as pltpu
from jax.experimental.pallas import tpu_sc as plsc

_NC, _NS = 2, 16
_NW = _NC * _NS
_CH = 40


def _worker_id():
    return lax.axis_index("s") * _NC + lax.axis_index("c")


def _sc_gather(nodes, idx3d, H):
    _, NCH, CH = idx3d.shape
    E = _NW * NCH * CH
    mesh = plsc.VectorSubcoreMesh(core_axis_name="c", subcore_axis_name="s")

    @functools.partial(
        pl.kernel,
        out_type=jax.ShapeDtypeStruct((E, H), jnp.float32),
        mesh=mesh,
        compiler_params=pltpu.CompilerParams(use_tc_tiling_on_sc=False),
        scratch_types=[
            pltpu.VMEM((NCH, CH), jnp.int32),
            pltpu.VMEM((CH, H), jnp.float32),
            pltpu.SemaphoreType.DMA,
        ],
    )
    def k(nodes_hbm, idx_hbm, out_hbm, idx_v, rows_v, sem):
        wid = _worker_id()
        pltpu.sync_copy(idx_hbm.at[wid], idx_v)

        def body(j, carry):
            pltpu.async_copy(nodes_hbm.at[idx_v.at[j]], rows_v, sem).wait()
            pltpu.sync_copy(rows_v, out_hbm.at[pl.ds((wid * NCH + j) * CH, CH)])
            return carry

        lax.fori_loop(0, NCH, body, 0)

    return k(nodes, idx3d)


def _sc_scatter_add(rows, idx3d, zeros_nh, H):
    _, NCH, CH = idx3d.shape
    NPAD = zeros_nh.shape[0]
    n_per_s = NPAD // _NS
    mesh = plsc.VectorSubcoreMesh(core_axis_name="c", subcore_axis_name="s")

    @functools.partial(
        pl.kernel,
        out_type=jax.ShapeDtypeStruct((_NC, NPAD, H), jnp.float32),
        mesh=mesh,
        compiler_params=pltpu.CompilerParams(use_tc_tiling_on_sc=False),
        scratch_types=[
            pltpu.VMEM((NCH, CH), jnp.int32),
            pltpu.VMEM((CH, H), jnp.float32),
            pltpu.VMEM_SHARED((NPAD, H), jnp.float32),
        ],
    )
    def k(rows_hbm, idx_hbm, zeros_hbm, out_hbm, idx_v, rows_v, acc):
        c = lax.axis_index("c")
        s = lax.axis_index("s")
        wid = s * _NC + c
        pltpu.sync_copy(zeros_hbm.at[pl.ds(s * n_per_s, n_per_s)],
                        acc.at[pl.ds(s * n_per_s, n_per_s)])
        pltpu.sync_copy(idx_hbm.at[wid], idx_v)
        plsc.subcore_barrier()

        def body(j, carry):
            pltpu.sync_copy(rows_hbm.at[pl.ds((wid * NCH + j) * CH, CH)],
                            rows_v)
            pltpu.sync_copy(rows_v, acc.at[idx_v.at[j]], add=True)
            return carry

        lax.fori_loop(0, NCH, body, 0)
        plsc.subcore_barrier()
        pltpu.sync_copy(acc.at[pl.ds(s * n_per_s, n_per_s)],
                        out_hbm.at[c, pl.ds(s * n_per_s, n_per_s)])

    return k(rows, idx3d, zeros_nh)


def _sc_degree_count(idx3d, ones_rows, zeros_nh, H):
    _, NCH, CH = idx3d.shape
    NPAD = zeros_nh.shape[0]
    n_per_s = NPAD // _NS
    mesh = plsc.VectorSubcoreMesh(core_axis_name="c", subcore_axis_name="s")

    @functools.partial(
        pl.kernel,
        out_type=jax.ShapeDtypeStruct((_NC, NPAD, H), jnp.float32),
        mesh=mesh,
        compiler_params=pltpu.CompilerParams(use_tc_tiling_on_sc=False),
        scratch_types=[
            pltpu.VMEM((NCH, CH), jnp.int32),
            pltpu.VMEM((CH, H), jnp.float32),
            pltpu.VMEM_SHARED((NPAD, H), jnp.float32),
        ],
    )
    def k(idx_hbm, ones_hbm, zeros_hbm, out_hbm, idx_v, rows_v, acc):
        c = lax.axis_index("c")
        s = lax.axis_index("s")
        wid = s * _NC + c
        pltpu.sync_copy(zeros_hbm.at[pl.ds(s * n_per_s, n_per_s)],
                        acc.at[pl.ds(s * n_per_s, n_per_s)])
        pltpu.sync_copy(idx_hbm.at[wid], idx_v)
        pltpu.sync_copy(ones_hbm, rows_v)
        plsc.subcore_barrier()

        def body(j, carry):
            pltpu.sync_copy(rows_v, acc.at[idx_v.at[j]], add=True)
            return carry

        lax.fori_loop(0, NCH, body, 0)
        plsc.subcore_barrier()
        pltpu.sync_copy(acc.at[pl.ds(s * n_per_s, n_per_s)],
                        out_hbm.at[c, pl.ds(s * n_per_s, n_per_s)])

    return k(idx3d, ones_rows, zeros_nh)


def _tc_input_proj(x, W0T, b0r):
    N = x.shape[0]
    H = W0T.shape[1]

    def body(x_ref, w_ref, b_ref, o_ref):
        o_ref[...] = jax.nn.relu(
            jnp.dot(x_ref[...], w_ref[...], preferred_element_type=jnp.float32)
            + b_ref[...])

    return pl.pallas_call(
        body, out_shape=jax.ShapeDtypeStruct((N, H), jnp.float32))(x, W0T, b0r)


def _tc_edge_mlp_t(ea_t, We1, be1c):
    K, E = We1.shape[0], ea_t.shape[1]
    EB = 16000
    grid = (E // EB,)

    def body(ea_ref, w_ref, b_ref, o_ref):
        f = jax.nn.relu(
            jnp.dot(w_ref[...], ea_ref[...], preferred_element_type=jnp.float32)
            + b_ref[...])
        o_ref[...] = f.astype(jnp.bfloat16)

    return pl.pallas_call(
        body,
        grid=grid,
        in_specs=[
            pl.BlockSpec((ea_t.shape[0], EB), lambda i: (0, i)),
            pl.BlockSpec(We1.shape, lambda i: (0, 0)),
            pl.BlockSpec(be1c.shape, lambda i: (0, 0)),
        ],
        out_specs=pl.BlockSpec((K, EB), lambda i: (0, i)),
        out_shape=jax.ShapeDtypeStruct((K, E), jnp.bfloat16),
    )(ea_t, We1, be1c)


def _tc_messages(f_t, xj, We2b, be2mT, H):
    K, E = f_t.shape
    EB = 640
    grid = (E // EB,)

    def body(ft_ref, xj_ref, w_ref, bm_ref, o_ref):
        wet = jnp.dot(w_ref[...], ft_ref[...],
                      preferred_element_type=jnp.float32)
        xjT = xj_ref[...].T
        msgT = jnp.dot(bm_ref[...], xjT, preferred_element_type=jnp.float32)
        for i in range(H):
            msgT = msgT + xjT[i:i + 1, :] * wet[i * H:(i + 1) * H, :]
        o_ref[...] = msgT.T

    return pl.pallas_call(
        body,
        grid=grid,
        in_specs=[
            pl.BlockSpec((K, EB), lambda i: (0, i)),
            pl.BlockSpec((EB, H), lambda i: (i, 0)),
            pl.BlockSpec(We2b.shape, lambda i: (0, 0)),
            pl.BlockSpec(be2mT.shape, lambda i: (0, 0)),
        ],
        out_specs=pl.BlockSpec((EB, H), lambda i: (i, 0)),
        out_shape=jax.ShapeDtypeStruct((E, H), jnp.float32),
    )(f_t, xj, We2b, be2mT)


def _tc_gru(parts, cparts, prev, rootm, biasr, gWihT, gWhhT, gbihr, gbhhr, H):
    N = prev.shape[0]
    NB = 1000
    grid = (N // NB,)

    def body(p_ref, c_ref, h_ref, root_ref, bias_ref, wih_ref, whh_ref,
             bih_ref, bhh_ref, o_ref):
        cnt = jnp.clip(c_ref[0, :, 0:1] + c_ref[1, :, 0:1], 1.0, None)
        agg = (p_ref[0] + p_ref[1]) / cnt
        h = h_ref[...]
        m = jax.nn.relu(
            agg + jnp.dot(h, root_ref[...], preferred_element_type=jnp.float32)
            + bias_ref[...])
        gi = jnp.dot(m, wih_ref[...], preferred_element_type=jnp.float32) + bih_ref[...]
        gh = jnp.dot(h, whh_ref[...], preferred_element_type=jnp.float32) + bhh_ref[...]
        r = jax.nn.sigmoid(gi[:, 0:H] + gh[:, 0:H])
        z = jax.nn.sigmoid(gi[:, H:2 * H] + gh[:, H:2 * H])
        n = jnp.tanh(gi[:, 2 * H:3 * H] + r * gh[:, 2 * H:3 * H])
        o_ref[...] = (1.0 - z) * n + z * h

    return pl.pallas_call(
        body,
        grid=grid,
        in_specs=[
            pl.BlockSpec((2, NB, H), lambda i: (0, i, 0)),
            pl.BlockSpec((2, NB, H), lambda i: (0, i, 0)),
            pl.BlockSpec((NB, H), lambda i: (i, 0)),
            pl.BlockSpec(rootm.shape, lambda i: (0, 0)),
            pl.BlockSpec(biasr.shape, lambda i: (0, 0)),
            pl.BlockSpec(gWihT.shape, lambda i: (0, 0)),
            pl.BlockSpec(gWhhT.shape, lambda i: (0, 0)),
            pl.BlockSpec(gbihr.shape, lambda i: (0, 0)),
            pl.BlockSpec(gbhhr.shape, lambda i: (0, 0)),
        ],
        out_specs=pl.BlockSpec((NB, H), lambda i: (i, 0)),
        out_shape=jax.ShapeDtypeStruct((N, H), jnp.float32),
    )(parts, cparts, prev, rootm, biasr, gWihT, gWhhT, gbihr, gbhhr)


def _tc_set2set(out, batch_col, sWihT, sWhhT, sbihr, sbhhr, W1T, b1r, W2T, b2r,
                B, H, PS):
    def body(out_ref, b_ref, wih_ref, whh_ref, bih_ref, bhh_ref,
             w1_ref, b1_ref, w2_ref, b2_ref, o_ref):
        nodes = out_ref[...]
        bcol = b_ref[...]
        onehot = (bcol == lax.broadcasted_iota(jnp.int32, (1, B), 1)
                  ).astype(jnp.float32)
        q_star = jnp.zeros((B, 2 * H), jnp.float32)
        hl = jnp.zeros((B, H), jnp.float32)
        cl = jnp.zeros((B, H), jnp.float32)
        for _ in range(PS):
            g = (jnp.dot(q_star, wih_ref[...], preferred_element_type=jnp.float32)
                 + bih_ref[...]
                 + jnp.dot(hl, whh_ref[...], preferred_element_type=jnp.float32)
                 + bhh_ref[...])
            cl = (jax.nn.sigmoid(g[:, H:2 * H]) * cl
                  + jax.nn.sigmoid(g[:, 0:H]) * jnp.tanh(g[:, 2 * H:3 * H]))
            hl = jax.nn.sigmoid(g[:, 3 * H:4 * H]) * jnp.tanh(cl)
            q_b = jnp.dot(onehot, hl, preferred_element_type=jnp.float32)
            e = jnp.sum(nodes * q_b, axis=1, keepdims=True)
            emax = jnp.max(jnp.where(onehot > 0.0, e, -jnp.inf), axis=0,
                           keepdims=True)
            emax_b = jnp.sum(onehot * emax, axis=1, keepdims=True)
            a = jnp.exp(e - emax_b)
            asum = jnp.sum(onehot * a, axis=0, keepdims=True)
            asum_b = jnp.sum(onehot * asum, axis=1, keepdims=True)
            a = a / (asum_b + 1e-16)
            r_ = lax.dot_general(onehot * a, nodes, (((0,), (0,)), ((), ())),
                                 preferred_element_type=jnp.float32)
            q_star = jnp.concatenate([hl, r_], axis=1)
        o = jax.nn.relu(
            jnp.dot(q_star, w1_ref[...], preferred_element_type=jnp.float32)
            + b1_ref[...])
        o_ref[...] = (jnp.dot(o, w2_ref[...], preferred_element_type=jnp.float32)
                      + b2_ref[...])

    return pl.pallas_call(
        body, out_shape=jax.ShapeDtypeStruct((B, 1), jnp.float32),
    )(out, batch_col, sWihT, sWhhT, sbihr, sbhhr, W1T, b1r, W2T, b2r)


def kernel(x, edge_index, edge_attr, batch, W0, b0, We1, be1, We2, be2, root,
           bias, gWih, gWhh, gbih, gbhh, sWih, sWhh, sbih, sbhh, W1, b1, W2,
           b2):
    N = x.shape[0]
    E = edge_index.shape[1]
    H = root.shape[0]
    B = 64
    L, PS = 3, 3
    NCH = E // (_NW * _CH)
    NPAD = ((N + 8 * _NS - 1) // (8 * _NS)) * (8 * _NS)

    src3d = edge_index[0].reshape(_NW, NCH, _CH)
    dst3d = edge_index[1].reshape(_NW, NCH, _CH)
    zeros_nh = jnp.zeros((NPAD, H), jnp.float32)
    ones_rows = jnp.ones((_CH, H), jnp.float32)
    batch_col = batch.reshape(N, 1)

    out = _tc_input_proj(x, W0.T, b0.reshape(1, H))
    f_t = _tc_edge_mlp_t(edge_attr.T, We1, be1.reshape(-1, 1))
    cparts = _sc_degree_count(dst3d, ones_rows, zeros_nh, H)

    We2b = We2.astype(jnp.bfloat16)
    be2mT = be2.reshape(H, H).T
    gWihT, gWhhT = gWih.T, gWhh.T
    gbihr, gbhhr = gbih.reshape(1, -1), gbhh.reshape(1, -1)
    biasr = bias.reshape(1, H)

    for _ in range(L):
        xj = _sc_gather(out, src3d, H)
        msg = _tc_messages(f_t, xj, We2b, be2mT, H)
        parts = _sc_scatter_add(msg, dst3d, zeros_nh, H)
        out = _tc_gru(parts, cparts, out, root, biasr, gWihT, gWhhT,
                      gbihr, gbhhr, H)

    o = _tc_set2set(out, batch_col, sWih.T, sWhh.T, sbih.reshape(1, -1),
                    sbhh.reshape(1, -1), W1.T, b1.reshape(1, H), W2.T,
                    b2.reshape(1, 1), B, H, PS)
    return o.reshape(-1)


# speedup vs baseline: 4.1926x; 1.6114x over previous
import functools

import jax
import jax.numpy as jnp
from jax import lax
from jax.experimental import pallas as pl
from jax.experimental.pallas import tpu as pltpu
from jax.experimental.pallas import tpu_sc as plsc

_NC, _NS = 2, 16
_NW = _NC * _NS
_CH = 40
_SEG = 25
_NSEG = 5
_SEG_E = _SEG * _CH

_sc_params = pltpu.CompilerParams(use_tc_tiling_on_sc=False)


def _mesh():
    return plsc.VectorSubcoreMesh(core_axis_name="c", subcore_axis_name="s")


def _sc_gather(nodes, idx1d, H):
    E = idx1d.shape[0]
    per_w = E // _NW

    @functools.partial(
        pl.kernel,
        out_type=jax.ShapeDtypeStruct((E, H), jnp.float32),
        mesh=_mesh(),
        compiler_params=_sc_params,
        scratch_types=[
            pltpu.VMEM((per_w,), jnp.int32),
            pltpu.VMEM((_SEG_E, H), jnp.float32),
            pltpu.VMEM((_SEG_E, H), jnp.float32),
            pltpu.VMEM((_SEG_E, H), jnp.float32),
            pltpu.SemaphoreType.DMA,
            pltpu.SemaphoreType.DMA,
            pltpu.SemaphoreType.DMA,
            pltpu.SemaphoreType.DMA,
            pltpu.SemaphoreType.DMA,
            pltpu.SemaphoreType.DMA,
        ],
    )
    def k(nodes_hbm, idx_hbm, out_hbm, idx_v, b0, b1, b2,
          g0, g1, g2, w0, w1, w2):
        bufs, gs, ws = [b0, b1, b2], [g0, g1, g2], [w0, w1, w2]
        wid = lax.axis_index("s") * _NC + lax.axis_index("c")
        base = wid * per_w
        pltpu.sync_copy(idx_hbm.at[pl.ds(base, per_w)], idx_v)
        for s in range(_NSEG):
            b = s % 3
            if s >= 3:
                pltpu.make_async_copy(out_hbm.at[pl.ds(0, _SEG_E)],
                                      bufs[b], ws[b]).wait()

            def issue(c, carry, _s=s, _b=b):
                off = (_s * _SEG + c) * _CH
                pltpu.async_copy(
                    nodes_hbm.at[idx_v.at[pl.ds(off, _CH)]],
                    bufs[_b].at[pl.ds(c * _CH, _CH)], gs[_b])
                return carry

            lax.fori_loop(0, _SEG, issue, 0)
            pltpu.make_async_copy(nodes_hbm.at[pl.ds(0, _SEG_E)],
                                  bufs[b], gs[b]).wait()
            pltpu.async_copy(bufs[b],
                             out_hbm.at[pl.ds(base + s * _SEG_E, _SEG_E)],
                             ws[b])
        for b in range(3):
            pltpu.make_async_copy(out_hbm.at[pl.ds(0, _SEG_E)],
                                  bufs[b], ws[b]).wait()

    return k(nodes, idx1d)


def _sc_scatter_add(rows, idx3d, zeros_nh, H):
    _, NCH, CH = idx3d.shape
    per_w = NCH * CH
    NPAD = zeros_nh.shape[0]
    n_per_s = NPAD // _NS

    @functools.partial(
        pl.kernel,
        out_type=jax.ShapeDtypeStruct((_NC, NPAD, H), jnp.float32),
        mesh=_mesh(),
        compiler_params=_sc_params,
        scratch_types=[
            pltpu.VMEM((NCH, CH), jnp.int32),
            pltpu.VMEM((_SEG_E, H), jnp.float32),
            pltpu.VMEM((_SEG_E, H), jnp.float32),
            pltpu.VMEM((_SEG_E, H), jnp.float32),
            pltpu.VMEM_SHARED((NPAD, H), jnp.float32),
            pltpu.SemaphoreType.DMA,
            pltpu.SemaphoreType.DMA,
            pltpu.SemaphoreType.DMA,
            pltpu.SemaphoreType.DMA,
            pltpu.SemaphoreType.DMA,
            pltpu.SemaphoreType.DMA,
        ],
    )
    def k(rows_hbm, idx_hbm, zeros_hbm, out_hbm, idx_v, b0, b1, b2, acc,
          r0, r1, r2, s0, s1, s2):
        bufs, rs, ss = [b0, b1, b2], [r0, r1, r2], [s0, s1, s2]
        c_ax = lax.axis_index("c")
        s_ax = lax.axis_index("s")
        wid = s_ax * _NC + c_ax
        base = wid * per_w
        pltpu.sync_copy(zeros_hbm.at[pl.ds(s_ax * n_per_s, n_per_s)],
                        acc.at[pl.ds(s_ax * n_per_s, n_per_s)])
        pltpu.sync_copy(idx_hbm.at[wid], idx_v)
        plsc.subcore_barrier()

        for s in range(3):
            pltpu.async_copy(rows_hbm.at[pl.ds(base + s * _SEG_E, _SEG_E)],
                             bufs[s], rs[s])
        for s in range(_NSEG):
            b = s % 3
            pltpu.make_async_copy(rows_hbm.at[pl.ds(0, _SEG_E)],
                                  bufs[b], rs[b]).wait()

            def issue(c, carry, _s=s, _b=b):
                j = _s * _SEG + c
                pltpu.async_copy(bufs[_b].at[pl.ds(c * _CH, _CH)],
                                 acc.at[idx_v.at[j]], ss[_b], add=True)
                return carry

            lax.fori_loop(0, _SEG, issue, 0)
            if s + 3 < _NSEG:
                pltpu.make_async_copy(rows_hbm.at[pl.ds(0, _SEG_E)],
                                      bufs[b], ss[b]).wait()
                pltpu.async_copy(
                    rows_hbm.at[pl.ds(base + (s + 3) * _SEG_E, _SEG_E)],
                    bufs[b], rs[b])
        for b in range(3):
            pltpu.make_async_copy(rows_hbm.at[pl.ds(0, _SEG_E)],
                                  bufs[b], ss[b]).wait()
        plsc.subcore_barrier()
        sl = pl.ds(s_ax * n_per_s, n_per_s)
        pltpu.sync_copy(acc.at[sl], out_hbm.at[c_ax, sl])

    return k(rows, idx3d, zeros_nh)


def _sc_degree_count(idx3d, ones_rows, zeros_nh, H):
    _, NCH, CH = idx3d.shape
    NPAD = zeros_nh.shape[0]
    n_per_s = NPAD // _NS

    @functools.partial(
        pl.kernel,
        out_type=jax.ShapeDtypeStruct((_NC, NPAD, H), jnp.float32),
        mesh=_mesh(),
        compiler_params=_sc_params,
        scratch_types=[
            pltpu.VMEM((NCH, CH), jnp.int32),
            pltpu.VMEM((CH, H), jnp.float32),
            pltpu.VMEM_SHARED((NPAD, H), jnp.float32),
            pltpu.SemaphoreType.DMA,
        ],
    )
    def k(idx_hbm, ones_hbm, zeros_hbm, out_hbm, idx_v, ones_v, acc, sem):
        c_ax = lax.axis_index("c")
        s_ax = lax.axis_index("s")
        wid = s_ax * _NC + c_ax
        pltpu.sync_copy(zeros_hbm.at[pl.ds(s_ax * n_per_s, n_per_s)],
                        acc.at[pl.ds(s_ax * n_per_s, n_per_s)])
        pltpu.sync_copy(idx_hbm.at[wid], idx_v)
        pltpu.sync_copy(ones_hbm, ones_v)
        plsc.subcore_barrier()

        def issue(j, carry):
            pltpu.async_copy(ones_v, acc.at[idx_v.at[j]], sem, add=True)
            return carry

        lax.fori_loop(0, NCH, issue, 0)

        def drain(j, carry):
            pltpu.make_async_copy(ones_hbm, ones_v, sem).wait()
            return carry

        lax.fori_loop(0, NCH, drain, 0)
        plsc.subcore_barrier()
        sl = pl.ds(s_ax * n_per_s, n_per_s)
        pltpu.sync_copy(acc.at[sl], out_hbm.at[c_ax, sl])

    return k(idx3d, ones_rows, zeros_nh)


def _tc_input_proj(x, W0T, b0r):
    N = x.shape[0]
    H = W0T.shape[1]

    def body(x_ref, w_ref, b_ref, o_ref):
        o_ref[...] = jax.nn.relu(
            jnp.dot(x_ref[...], w_ref[...], preferred_element_type=jnp.float32)
            + b_ref[...])

    return pl.pallas_call(
        body, out_shape=jax.ShapeDtypeStruct((N, H), jnp.float32))(x, W0T, b0r)


def _tc_edge_mlp_t(ea_t, We1, be1c):
    K, E = We1.shape[0], ea_t.shape[1]
    EB = 16000
    grid = (E // EB,)

    def body(ea_ref, w_ref, b_ref, o_ref):
        f = jax.nn.relu(
            jnp.dot(w_ref[...], ea_ref[...], preferred_element_type=jnp.float32)
            + b_ref[...])
        o_ref[...] = f.astype(jnp.bfloat16)

    return pl.pallas_call(
        body,
        grid=grid,
        in_specs=[
            pl.BlockSpec((ea_t.shape[0], EB), lambda i: (0, i)),
            pl.BlockSpec(We1.shape, lambda i: (0, 0)),
            pl.BlockSpec(be1c.shape, lambda i: (0, 0)),
        ],
        out_specs=pl.BlockSpec((K, EB), lambda i: (0, i)),
        out_shape=jax.ShapeDtypeStruct((K, E), jnp.bfloat16),
    )(ea_t, We1, be1c)


def _tc_messages(f_t, xj, We2b, be2mT, H):
    K, E = f_t.shape
    EB = 1280
    HB = EB // 2
    grid = (E // EB,)

    def body(ft_ref, xj_ref, w_ref, bm_ref, o_ref):
        xjT = xj_ref[...].T
        bias = jnp.dot(bm_ref[...], xjT, preferred_element_type=jnp.float32)
        halves = []
        for h in range(2):
            sl = slice(h * HB, (h + 1) * HB)
            wet = jnp.dot(w_ref[...], ft_ref[:, sl],
                          preferred_element_type=jnp.float32)
            acc = bias[:, sl]
            for i in range(H):
                acc = acc + xjT[i:i + 1, sl] * wet[i * H:(i + 1) * H, :]
            halves.append(acc)
        o_ref[...] = jnp.concatenate(halves, axis=1).T

    return pl.pallas_call(
        body,
        grid=grid,
        in_specs=[
            pl.BlockSpec((K, EB), lambda i: (0, i)),
            pl.BlockSpec((EB, H), lambda i: (i, 0)),
            pl.BlockSpec(We2b.shape, lambda i: (0, 0)),
            pl.BlockSpec(be2mT.shape, lambda i: (0, 0)),
        ],
        out_specs=pl.BlockSpec((EB, H), lambda i: (i, 0)),
        out_shape=jax.ShapeDtypeStruct((E, H), jnp.float32),
    )(f_t, xj, We2b, be2mT)


def _tc_gru(parts, cparts, prev, rootm, biasr, gWihT, gWhhT, gbihr, gbhhr, H):
    N = prev.shape[0]
    NB = 1000
    grid = (N // NB,)

    def body(p_ref, c_ref, h_ref, root_ref, bias_ref, wih_ref, whh_ref,
             bih_ref, bhh_ref, o_ref):
        cnt = jnp.clip(c_ref[0, :, 0:1] + c_ref[1, :, 0:1], 1.0, None)
        agg = (p_ref[0] + p_ref[1]) / cnt
        h = h_ref[...]
        m = jax.nn.relu(
            agg + jnp.dot(h, root_ref[...], preferred_element_type=jnp.float32)
            + bias_ref[...])
        gi = jnp.dot(m, wih_ref[...], preferred_element_type=jnp.float32) + bih_ref[...]
        gh = jnp.dot(h, whh_ref[...], preferred_element_type=jnp.float32) + bhh_ref[...]
        r = jax.nn.sigmoid(gi[:, 0:H] + gh[:, 0:H])
        z = jax.nn.sigmoid(gi[:, H:2 * H] + gh[:, H:2 * H])
        n = jnp.tanh(gi[:, 2 * H:3 * H] + r * gh[:, 2 * H:3 * H])
        o_ref[...] = (1.0 - z) * n + z * h

    return pl.pallas_call(
        body,
        grid=grid,
        in_specs=[
            pl.BlockSpec((2, NB, H), lambda i: (0, i, 0)),
            pl.BlockSpec((2, NB, H), lambda i: (0, i, 0)),
            pl.BlockSpec((NB, H), lambda i: (i, 0)),
            pl.BlockSpec(rootm.shape, lambda i: (0, 0)),
            pl.BlockSpec(biasr.shape, lambda i: (0, 0)),
            pl.BlockSpec(gWihT.shape, lambda i: (0, 0)),
            pl.BlockSpec(gWhhT.shape, lambda i: (0, 0)),
            pl.BlockSpec(gbihr.shape, lambda i: (0, 0)),
            pl.BlockSpec(gbhhr.shape, lambda i: (0, 0)),
        ],
        out_specs=pl.BlockSpec((NB, H), lambda i: (i, 0)),
        out_shape=jax.ShapeDtypeStruct((N, H), jnp.float32),
    )(parts, cparts, prev, rootm, biasr, gWihT, gWhhT, gbihr, gbhhr)


def _tc_set2set(out, batch_col, sWihT, sWhhT, sbihr, sbhhr, W1T, b1r, W2T, b2r,
                B, H, PS):
    def body(out_ref, b_ref, wih_ref, whh_ref, bih_ref, bhh_ref,
             w1_ref, b1_ref, w2_ref, b2_ref, o_ref):
        nodes = out_ref[...]
        bcol = b_ref[...]
        onehot = (bcol == lax.broadcasted_iota(jnp.int32, (1, B), 1)
                  ).astype(jnp.float32)
        q_star = jnp.zeros((B, 2 * H), jnp.float32)
        hl = jnp.zeros((B, H), jnp.float32)
        cl = jnp.zeros((B, H), jnp.float32)
        for _ in range(PS):
            g = (jnp.dot(q_star, wih_ref[...], preferred_element_type=jnp.float32)
                 + bih_ref[...]
                 + jnp.dot(hl, whh_ref[...], preferred_element_type=jnp.float32)
                 + bhh_ref[...])
            cl = (jax.nn.sigmoid(g[:, H:2 * H]) * cl
                  + jax.nn.sigmoid(g[:, 0:H]) * jnp.tanh(g[:, 2 * H:3 * H]))
            hl = jax.nn.sigmoid(g[:, 3 * H:4 * H]) * jnp.tanh(cl)
            q_b = jnp.dot(onehot, hl, preferred_element_type=jnp.float32)
            e = jnp.sum(nodes * q_b, axis=1, keepdims=True)
            emax = jnp.max(jnp.where(onehot > 0.0, e, -jnp.inf), axis=0,
                           keepdims=True)
            emax_b = jnp.sum(onehot * emax, axis=1, keepdims=True)
            a = jnp.exp(e - emax_b)
            asum = jnp.sum(onehot * a, axis=0, keepdims=True)
            asum_b = jnp.sum(onehot * asum, axis=1, keepdims=True)
            a = a / (asum_b + 1e-16)
            r_ = lax.dot_general(onehot * a, nodes, (((0,), (0,)), ((), ())),
                                 preferred_element_type=jnp.float32)
            q_star = jnp.concatenate([hl, r_], axis=1)
        o = jax.nn.relu(
            jnp.dot(q_star, w1_ref[...], preferred_element_type=jnp.float32)
            + b1_ref[...])
        o_ref[...] = (jnp.dot(o, w2_ref[...], preferred_element_type=jnp.float32)
                      + b2_ref[...])

    return pl.pallas_call(
        body, out_shape=jax.ShapeDtypeStruct((B, 1), jnp.float32),
    )(out, batch_col, sWihT, sWhhT, sbihr, sbhhr, W1T, b1r, W2T, b2r)


def kernel(x, edge_index, edge_attr, batch, W0, b0, We1, be1, We2, be2, root,
           bias, gWih, gWhh, gbih, gbhh, sWih, sWhh, sbih, sbhh, W1, b1, W2,
           b2):
    N = x.shape[0]
    E = edge_index.shape[1]
    H = root.shape[0]
    B = 64
    L, PS = 3, 3
    NCH = E // (_NW * _CH)
    NPAD = ((N + 8 * _NS - 1) // (8 * _NS)) * (8 * _NS)

    src1d = edge_index[0]
    dst3d = edge_index[1].reshape(_NW, NCH, _CH)
    zeros_nh = jnp.zeros((NPAD, H), jnp.float32)
    ones_rows = jnp.ones((_CH, H), jnp.float32)
    batch_col = batch.reshape(N, 1)

    out = _tc_input_proj(x, W0.T, b0.reshape(1, H))
    f_t = _tc_edge_mlp_t(edge_attr.T, We1, be1.reshape(-1, 1))

    We2b = We2.astype(jnp.bfloat16)
    be2mT = be2.reshape(H, H).T
    gWihT, gWhhT = gWih.T, gWhh.T
    gbihr, gbhhr = gbih.reshape(1, -1), gbhh.reshape(1, -1)
    biasr = bias.reshape(1, H)

    cparts = _sc_degree_count(dst3d, ones_rows, zeros_nh, H)
    for layer in range(L):
        xj = _sc_gather(out, src1d, H)
        msg = _tc_messages(f_t, xj, We2b, be2mT, H)
        parts = _sc_scatter_add(msg, dst3d, zeros_nh, H)
        out = _tc_gru(parts, cparts, out, root, biasr, gWihT, gWhhT,
                      gbihr, gbhhr, H)

    o = _tc_set2set(out, batch_col, sWih.T, sWhh.T, sbih.reshape(1, -1),
                    sbhh.reshape(1, -1), W1.T, b1.reshape(1, H), W2.T,
                    b2.reshape(1, 1), B, H, PS)
    return o.reshape(-1)


# speedup vs baseline: 4.8102x; 1.1473x over previous
import functools

import jax
import jax.numpy as jnp
from jax import lax
from jax.experimental import pallas as pl
from jax.experimental.pallas import tpu as pltpu
from jax.experimental.pallas import tpu_sc as plsc

_NC, _NS = 2, 16
_NW = _NC * _NS
_CH = 40
_SEG = 25
_NSEG = 5
_SEG_E = _SEG * _CH

_sc_params = pltpu.CompilerParams(use_tc_tiling_on_sc=False)


def _mesh():
    return plsc.VectorSubcoreMesh(core_axis_name="c", subcore_axis_name="s")


def _sc_gather(nodes, idx1d, H):
    E = idx1d.shape[0]
    per_w = E // _NW

    @functools.partial(
        pl.kernel,
        out_type=jax.ShapeDtypeStruct((E, H), jnp.float32),
        mesh=_mesh(),
        compiler_params=_sc_params,
        scratch_types=[
            pltpu.VMEM((per_w,), jnp.int32),
            pltpu.VMEM((_SEG_E, H), jnp.float32),
            pltpu.VMEM((_SEG_E, H), jnp.float32),
            pltpu.VMEM((_SEG_E, H), jnp.float32),
            pltpu.SemaphoreType.DMA,
            pltpu.SemaphoreType.DMA,
            pltpu.SemaphoreType.DMA,
            pltpu.SemaphoreType.DMA,
            pltpu.SemaphoreType.DMA,
            pltpu.SemaphoreType.DMA,
        ],
    )
    def k(nodes_hbm, idx_hbm, out_hbm, idx_v, b0, b1, b2,
          g0, g1, g2, w0, w1, w2):
        bufs, gs, ws = [b0, b1, b2], [g0, g1, g2], [w0, w1, w2]
        wid = lax.axis_index("s") * _NC + lax.axis_index("c")
        base = wid * per_w
        pltpu.sync_copy(idx_hbm.at[pl.ds(base, per_w)], idx_v)
        for s in range(_NSEG):
            b = s % 3
            if s >= 3:
                pltpu.make_async_copy(out_hbm.at[pl.ds(0, _SEG_E)],
                                      bufs[b], ws[b]).wait()

            def issue(c, carry, _s=s, _b=b):
                off = (_s * _SEG + c) * _CH
                pltpu.async_copy(
                    nodes_hbm.at[idx_v.at[pl.ds(off, _CH)]],
                    bufs[_b].at[pl.ds(c * _CH, _CH)], gs[_b])
                return carry

            lax.fori_loop(0, _SEG, issue, 0)
            pltpu.make_async_copy(nodes_hbm.at[pl.ds(0, _SEG_E)],
                                  bufs[b], gs[b]).wait()
            pltpu.async_copy(bufs[b],
                             out_hbm.at[pl.ds(base + s * _SEG_E, _SEG_E)],
                             ws[b])
        for b in range(3):
            pltpu.make_async_copy(out_hbm.at[pl.ds(0, _SEG_E)],
                                  bufs[b], ws[b]).wait()

    return k(nodes, idx1d)


def _sc_scatter_add(rows, idx3d, zeros_nh, H):
    _, NCH, CH = idx3d.shape
    per_w = NCH * CH
    NPAD = zeros_nh.shape[0]
    n_per_s = NPAD // _NS

    @functools.partial(
        pl.kernel,
        out_type=jax.ShapeDtypeStruct((_NC, NPAD, H), jnp.float32),
        mesh=_mesh(),
        compiler_params=_sc_params,
        scratch_types=[
            pltpu.VMEM((NCH, CH), jnp.int32),
            pltpu.VMEM((_SEG_E, H), jnp.float32),
            pltpu.VMEM((_SEG_E, H), jnp.float32),
            pltpu.VMEM((_SEG_E, H), jnp.float32),
            pltpu.VMEM_SHARED((NPAD, H), jnp.float32),
            pltpu.SemaphoreType.DMA,
            pltpu.SemaphoreType.DMA,
            pltpu.SemaphoreType.DMA,
            pltpu.SemaphoreType.DMA,
            pltpu.SemaphoreType.DMA,
            pltpu.SemaphoreType.DMA,
        ],
    )
    def k(rows_hbm, idx_hbm, zeros_hbm, out_hbm, idx_v, b0, b1, b2, acc,
          r0, r1, r2, s0, s1, s2):
        bufs, rs, ss = [b0, b1, b2], [r0, r1, r2], [s0, s1, s2]
        c_ax = lax.axis_index("c")
        s_ax = lax.axis_index("s")
        wid = s_ax * _NC + c_ax
        base = wid * per_w
        pltpu.sync_copy(zeros_hbm.at[pl.ds(s_ax * n_per_s, n_per_s)],
                        acc.at[pl.ds(s_ax * n_per_s, n_per_s)])
        pltpu.sync_copy(idx_hbm.at[wid], idx_v)
        plsc.subcore_barrier()

        for s in range(3):
            pltpu.async_copy(rows_hbm.at[pl.ds(base + s * _SEG_E, _SEG_E)],
                             bufs[s], rs[s])
        for s in range(_NSEG):
            b = s % 3
            pltpu.make_async_copy(rows_hbm.at[pl.ds(0, _SEG_E)],
                                  bufs[b], rs[b]).wait()

            def issue(c, carry, _s=s, _b=b):
                j = _s * _SEG + c
                pltpu.async_copy(bufs[_b].at[pl.ds(c * _CH, _CH)],
                                 acc.at[idx_v.at[j]], ss[_b], add=True)
                return carry

            lax.fori_loop(0, _SEG, issue, 0)
            if s + 3 < _NSEG:
                pltpu.make_async_copy(rows_hbm.at[pl.ds(0, _SEG_E)],
                                      bufs[b], ss[b]).wait()
                pltpu.async_copy(
                    rows_hbm.at[pl.ds(base + (s + 3) * _SEG_E, _SEG_E)],
                    bufs[b], rs[b])
        for b in range(3):
            pltpu.make_async_copy(rows_hbm.at[pl.ds(0, _SEG_E)],
                                  bufs[b], ss[b]).wait()
        plsc.subcore_barrier()
        sl = pl.ds(s_ax * n_per_s, n_per_s)
        pltpu.sync_copy(acc.at[sl], out_hbm.at[c_ax, sl])

    return k(rows, idx3d, zeros_nh)


def _sc_degree_count(idx3d, ones_rows, zeros_nh, H):
    _, NCH, CH = idx3d.shape
    NPAD = zeros_nh.shape[0]
    n_per_s = NPAD // _NS

    @functools.partial(
        pl.kernel,
        out_type=jax.ShapeDtypeStruct((_NC, NPAD, H), jnp.float32),
        mesh=_mesh(),
        compiler_params=_sc_params,
        scratch_types=[
            pltpu.VMEM((NCH, CH), jnp.int32),
            pltpu.VMEM((CH, H), jnp.float32),
            pltpu.VMEM_SHARED((NPAD, H), jnp.float32),
            pltpu.SemaphoreType.DMA,
        ],
    )
    def k(idx_hbm, ones_hbm, zeros_hbm, out_hbm, idx_v, ones_v, acc, sem):
        c_ax = lax.axis_index("c")
        s_ax = lax.axis_index("s")
        wid = s_ax * _NC + c_ax
        pltpu.sync_copy(zeros_hbm.at[pl.ds(s_ax * n_per_s, n_per_s)],
                        acc.at[pl.ds(s_ax * n_per_s, n_per_s)])
        pltpu.sync_copy(idx_hbm.at[wid], idx_v)
        pltpu.sync_copy(ones_hbm, ones_v)
        plsc.subcore_barrier()

        def issue(j, carry):
            pltpu.async_copy(ones_v, acc.at[idx_v.at[j]], sem, add=True)
            return carry

        lax.fori_loop(0, NCH, issue, 0)

        def drain(j, carry):
            pltpu.make_async_copy(ones_hbm, ones_v, sem).wait()
            return carry

        lax.fori_loop(0, NCH, drain, 0)
        plsc.subcore_barrier()
        sl = pl.ds(s_ax * n_per_s, n_per_s)
        pltpu.sync_copy(acc.at[sl], out_hbm.at[c_ax, sl])

    return k(idx3d, ones_rows, zeros_nh)


def _tc_input_proj(x, W0T, b0r):
    N = x.shape[0]
    H = W0T.shape[1]

    def body(x_ref, w_ref, b_ref, o_ref):
        o_ref[...] = jax.nn.relu(
            jnp.dot(x_ref[...], w_ref[...], preferred_element_type=jnp.float32)
            + b_ref[...])

    return pl.pallas_call(
        body, out_shape=jax.ShapeDtypeStruct((N, H), jnp.float32))(x, W0T, b0r)


def _tc_edge_mlp_t(ea_t, We1, be1c):
    K, E = We1.shape[0], ea_t.shape[1]
    EB = 16000
    grid = (E // EB,)

    def body(ea_ref, w_ref, b_ref, o_ref):
        f = jax.nn.relu(
            jnp.dot(w_ref[...], ea_ref[...], preferred_element_type=jnp.float32)
            + b_ref[...])
        o_ref[...] = f.astype(jnp.bfloat16)

    return pl.pallas_call(
        body,
        grid=grid,
        in_specs=[
            pl.BlockSpec((ea_t.shape[0], EB), lambda i: (0, i)),
            pl.BlockSpec(We1.shape, lambda i: (0, 0)),
            pl.BlockSpec(be1c.shape, lambda i: (0, 0)),
        ],
        out_specs=pl.BlockSpec((K, EB), lambda i: (0, i)),
        out_shape=jax.ShapeDtypeStruct((K, E), jnp.bfloat16),
    )(ea_t, We1, be1c)


def _tc_messages(f_t, xj, We2b, be2mT, H):
    K, E = f_t.shape
    EB = 3200
    NHALF = 5
    HB = EB // NHALF
    grid = (E // EB,)

    def body(ft_ref, xj_ref, w_ref, bm_ref, o_ref):
        xjT = xj_ref[...].T
        bias = jnp.dot(bm_ref[...], xjT, preferred_element_type=jnp.float32)
        halves = []
        for h in range(NHALF):
            sl = slice(h * HB, (h + 1) * HB)
            wet = jnp.dot(w_ref[...], ft_ref[:, sl],
                          preferred_element_type=jnp.float32)
            acc = bias[:, sl]
            for i in range(H):
                acc = acc + xjT[i:i + 1, sl] * wet[i * H:(i + 1) * H, :]
            halves.append(acc)
        o_ref[...] = jnp.concatenate(halves, axis=1).T

    return pl.pallas_call(
        body,
        grid=grid,
        in_specs=[
            pl.BlockSpec((K, EB), lambda i: (0, i)),
            pl.BlockSpec((EB, H), lambda i: (i, 0)),
            pl.BlockSpec(We2b.shape, lambda i: (0, 0)),
            pl.BlockSpec(be2mT.shape, lambda i: (0, 0)),
        ],
        out_specs=pl.BlockSpec((EB, H), lambda i: (i, 0)),
        out_shape=jax.ShapeDtypeStruct((E, H), jnp.float32),
    )(f_t, xj, We2b, be2mT)


def _tc_gru(parts, cparts, prev, rootm, biasr, gWihT, gWhhT, gbihr, gbhhr, H):
    N = prev.shape[0]
    NB = 1000
    grid = (N // NB,)

    def body(p_ref, c_ref, h_ref, root_ref, bias_ref, wih_ref, whh_ref,
             bih_ref, bhh_ref, o_ref):
        cnt = jnp.clip(c_ref[0, :, 0:1] + c_ref[1, :, 0:1], 1.0, None)
        agg = (p_ref[0] + p_ref[1]) / cnt
        h = h_ref[...]
        m = jax.nn.relu(
            agg + jnp.dot(h, root_ref[...], preferred_element_type=jnp.float32)
            + bias_ref[...])
        gi = jnp.dot(m, wih_ref[...], preferred_element_type=jnp.float32) + bih_ref[...]
        gh = jnp.dot(h, whh_ref[...], preferred_element_type=jnp.float32) + bhh_ref[...]
        r = jax.nn.sigmoid(gi[:, 0:H] + gh[:, 0:H])
        z = jax.nn.sigmoid(gi[:, H:2 * H] + gh[:, H:2 * H])
        n = jnp.tanh(gi[:, 2 * H:3 * H] + r * gh[:, 2 * H:3 * H])
        o_ref[...] = (1.0 - z) * n + z * h

    return pl.pallas_call(
        body,
        grid=grid,
        in_specs=[
            pl.BlockSpec((2, NB, H), lambda i: (0, i, 0)),
            pl.BlockSpec((2, NB, H), lambda i: (0, i, 0)),
            pl.BlockSpec((NB, H), lambda i: (i, 0)),
            pl.BlockSpec(rootm.shape, lambda i: (0, 0)),
            pl.BlockSpec(biasr.shape, lambda i: (0, 0)),
            pl.BlockSpec(gWihT.shape, lambda i: (0, 0)),
            pl.BlockSpec(gWhhT.shape, lambda i: (0, 0)),
            pl.BlockSpec(gbihr.shape, lambda i: (0, 0)),
            pl.BlockSpec(gbhhr.shape, lambda i: (0, 0)),
        ],
        out_specs=pl.BlockSpec((NB, H), lambda i: (i, 0)),
        out_shape=jax.ShapeDtypeStruct((N, H), jnp.float32),
    )(parts, cparts, prev, rootm, biasr, gWihT, gWhhT, gbihr, gbhhr)


def _tc_set2set(out, batch_col, sWihT, sWhhT, sbihr, sbhhr, W1T, b1r, W2T, b2r,
                B, H, PS):
    def body(out_ref, b_ref, wih_ref, whh_ref, bih_ref, bhh_ref,
             w1_ref, b1_ref, w2_ref, b2_ref, o_ref):
        nodes = out_ref[...]
        bcol = b_ref[...]
        onehot = (bcol == lax.broadcasted_iota(jnp.int32, (1, B), 1)
                  ).astype(jnp.float32)
        q_star = jnp.zeros((B, 2 * H), jnp.float32)
        hl = jnp.zeros((B, H), jnp.float32)
        cl = jnp.zeros((B, H), jnp.float32)
        for _ in range(PS):
            g = (jnp.dot(q_star, wih_ref[...], preferred_element_type=jnp.float32)
                 + bih_ref[...]
                 + jnp.dot(hl, whh_ref[...], preferred_element_type=jnp.float32)
                 + bhh_ref[...])
            cl = (jax.nn.sigmoid(g[:, H:2 * H]) * cl
                  + jax.nn.sigmoid(g[:, 0:H]) * jnp.tanh(g[:, 2 * H:3 * H]))
            hl = jax.nn.sigmoid(g[:, 3 * H:4 * H]) * jnp.tanh(cl)
            q_b = jnp.dot(onehot, hl, preferred_element_type=jnp.float32)
            e = jnp.sum(nodes * q_b, axis=1, keepdims=True)
            emax = jnp.max(jnp.where(onehot > 0.0, e, -jnp.inf), axis=0,
                           keepdims=True)
            emax_b = jnp.sum(onehot * emax, axis=1, keepdims=True)
            a = jnp.exp(e - emax_b)
            asum = jnp.sum(onehot * a, axis=0, keepdims=True)
            asum_b = jnp.sum(onehot * asum, axis=1, keepdims=True)
            a = a / (asum_b + 1e-16)
            r_ = lax.dot_general(onehot * a, nodes, (((0,), (0,)), ((), ())),
                                 preferred_element_type=jnp.float32)
            q_star = jnp.concatenate([hl, r_], axis=1)
        o = jax.nn.relu(
            jnp.dot(q_star, w1_ref[...], preferred_element_type=jnp.float32)
            + b1_ref[...])
        o_ref[...] = (jnp.dot(o, w2_ref[...], preferred_element_type=jnp.float32)
                      + b2_ref[...])

    return pl.pallas_call(
        body, out_shape=jax.ShapeDtypeStruct((B, 1), jnp.float32),
    )(out, batch_col, sWihT, sWhhT, sbihr, sbhhr, W1T, b1r, W2T, b2r)


def kernel(x, edge_index, edge_attr, batch, W0, b0, We1, be1, We2, be2, root,
           bias, gWih, gWhh, gbih, gbhh, sWih, sWhh, sbih, sbhh, W1, b1, W2,
           b2):
    N = x.shape[0]
    E = edge_index.shape[1]
    H = root.shape[0]
    B = 64
    L, PS = 3, 3
    NCH = E // (_NW * _CH)
    NPAD = ((N + 8 * _NS - 1) // (8 * _NS)) * (8 * _NS)

    src1d = edge_index[0]
    dst3d = edge_index[1].reshape(_NW, NCH, _CH)
    zeros_nh = jnp.zeros((NPAD, H), jnp.float32)
    ones_rows = jnp.ones((_CH, H), jnp.float32)
    batch_col = batch.reshape(N, 1)

    out = _tc_input_proj(x, W0.T, b0.reshape(1, H))
    f_t = _tc_edge_mlp_t(edge_attr.T, We1, be1.reshape(-1, 1))

    We2b = We2.astype(jnp.bfloat16)
    be2mT = be2.reshape(H, H).T
    gWihT, gWhhT = gWih.T, gWhh.T
    gbihr, gbhhr = gbih.reshape(1, -1), gbhh.reshape(1, -1)
    biasr = bias.reshape(1, H)

    cparts = _sc_degree_count(dst3d, ones_rows, zeros_nh, H)
    for layer in range(L):
        xj = _sc_gather(out, src1d, H)
        msg = _tc_messages(f_t, xj, We2b, be2mT, H)
        parts = _sc_scatter_add(msg, dst3d, zeros_nh, H)
        out = _tc_gru(parts, cparts, out, root, biasr, gWihT, gWhhT,
                      gbihr, gbhhr, H)

    o = _tc_set2set(out, batch_col, sWih.T, sWhh.T, sbih.reshape(1, -1),
                    sbhh.reshape(1, -1), W1.T, b1.reshape(1, H), W2.T,
                    b2.reshape(1, 1), B, H, PS)
    return o.reshape(-1)


# speedup vs baseline: 7.0698x; 1.4698x over previous
import functools

import jax
import jax.numpy as jnp
from jax import lax
from jax.experimental import pallas as pl
from jax.experimental.pallas import tpu as pltpu
from jax.experimental.pallas import tpu_sc as plsc

_NC, _NS = 2, 16
_NW = _NC * _NS
_CH = 40
_SEG = 25
_NSEG = 5
_SEG_E = _SEG * _CH

_sc_params = pltpu.CompilerParams(use_tc_tiling_on_sc=False)


def _mesh():
    return plsc.VectorSubcoreMesh(core_axis_name="c", subcore_axis_name="s")


def _sc_gather(nodes, idx1d, H):
    E = idx1d.shape[0]
    per_w = E // _NW

    @functools.partial(
        pl.kernel,
        out_type=jax.ShapeDtypeStruct((E, 128), jnp.float32),
        mesh=_mesh(),
        compiler_params=_sc_params,
        scratch_types=[
            pltpu.VMEM((per_w,), jnp.int32),
            pltpu.VMEM((_SEG_E, H), jnp.float32),
            pltpu.VMEM((_SEG_E, H), jnp.float32),
            pltpu.VMEM((_SEG_E, H), jnp.float32),
            pltpu.SemaphoreType.DMA,
            pltpu.SemaphoreType.DMA,
            pltpu.SemaphoreType.DMA,
            pltpu.SemaphoreType.DMA,
            pltpu.SemaphoreType.DMA,
            pltpu.SemaphoreType.DMA,
        ],
    )
    def k(nodes_hbm, idx_hbm, out_hbm, idx_v, b0, b1, b2,
          g0, g1, g2, w0, w1, w2):
        bufs, gs, ws = [b0, b1, b2], [g0, g1, g2], [w0, w1, w2]
        wid = lax.axis_index("s") * _NC + lax.axis_index("c")
        base = wid * per_w
        pltpu.sync_copy(idx_hbm.at[pl.ds(base, per_w)], idx_v)
        for s in range(_NSEG):
            b = s % 3
            if s >= 3:
                pltpu.make_async_copy(out_hbm.at[pl.ds(0, _SEG_E),
                                                 pl.ds(0, H)],
                                      bufs[b], ws[b]).wait()

            def issue(c, carry, _s=s, _b=b):
                off = (_s * _SEG + c) * _CH
                pltpu.async_copy(
                    nodes_hbm.at[idx_v.at[pl.ds(off, _CH)]],
                    bufs[_b].at[pl.ds(c * _CH, _CH)], gs[_b])
                return carry

            lax.fori_loop(0, _SEG, issue, 0)
            pltpu.make_async_copy(nodes_hbm.at[pl.ds(0, _SEG_E)],
                                  bufs[b], gs[b]).wait()
            pltpu.async_copy(bufs[b],
                             out_hbm.at[pl.ds(base + s * _SEG_E, _SEG_E),
                                        pl.ds(0, H)],
                             ws[b])
        for b in range(3):
            pltpu.make_async_copy(out_hbm.at[pl.ds(0, _SEG_E), pl.ds(0, H)],
                                  bufs[b], ws[b]).wait()

    return k(nodes, idx1d)


def _sc_scatter_add(rows, idx3d, zeros_nh, H):
    _, NCH, CH = idx3d.shape
    per_w = NCH * CH
    NPAD = zeros_nh.shape[0]
    n_per_s = NPAD // _NS

    @functools.partial(
        pl.kernel,
        out_type=jax.ShapeDtypeStruct((_NC, NPAD, H), jnp.float32),
        mesh=_mesh(),
        compiler_params=_sc_params,
        scratch_types=[
            pltpu.VMEM((NCH, CH), jnp.int32),
            pltpu.VMEM((_SEG_E, H), jnp.float32),
            pltpu.VMEM((_SEG_E, H), jnp.float32),
            pltpu.VMEM((_SEG_E, H), jnp.float32),
            pltpu.VMEM_SHARED((NPAD, H), jnp.float32),
            pltpu.SemaphoreType.DMA,
            pltpu.SemaphoreType.DMA,
            pltpu.SemaphoreType.DMA,
            pltpu.SemaphoreType.DMA,
            pltpu.SemaphoreType.DMA,
            pltpu.SemaphoreType.DMA,
        ],
    )
    def k(rows_hbm, idx_hbm, zeros_hbm, out_hbm, idx_v, b0, b1, b2, acc,
          r0, r1, r2, s0, s1, s2):
        bufs, rs, ss = [b0, b1, b2], [r0, r1, r2], [s0, s1, s2]
        c_ax = lax.axis_index("c")
        s_ax = lax.axis_index("s")
        wid = s_ax * _NC + c_ax
        base = wid * per_w
        pltpu.sync_copy(zeros_hbm.at[pl.ds(s_ax * n_per_s, n_per_s)],
                        acc.at[pl.ds(s_ax * n_per_s, n_per_s)])
        pltpu.sync_copy(idx_hbm.at[wid], idx_v)
        plsc.subcore_barrier()

        for s in range(3):
            pltpu.async_copy(
                rows_hbm.at[pl.ds(base + s * _SEG_E, _SEG_E), pl.ds(0, H)],
                bufs[s], rs[s])
        for s in range(_NSEG):
            b = s % 3
            pltpu.make_async_copy(rows_hbm.at[pl.ds(0, _SEG_E), pl.ds(0, H)],
                                  bufs[b], rs[b]).wait()

            def issue(c, carry, _s=s, _b=b):
                j = _s * _SEG + c
                pltpu.async_copy(bufs[_b].at[pl.ds(c * _CH, _CH)],
                                 acc.at[idx_v.at[j]], ss[_b], add=True)
                return carry

            lax.fori_loop(0, _SEG, issue, 0)
            if s + 3 < _NSEG:
                pltpu.make_async_copy(rows_hbm.at[pl.ds(0, _SEG_E),
                                                  pl.ds(0, H)],
                                      bufs[b], ss[b]).wait()
                pltpu.async_copy(
                    rows_hbm.at[pl.ds(base + (s + 3) * _SEG_E, _SEG_E),
                                pl.ds(0, H)],
                    bufs[b], rs[b])
        for b in range(3):
            pltpu.make_async_copy(rows_hbm.at[pl.ds(0, _SEG_E), pl.ds(0, H)],
                                  bufs[b], ss[b]).wait()
        plsc.subcore_barrier()
        sl = pl.ds(s_ax * n_per_s, n_per_s)
        pltpu.sync_copy(acc.at[sl], out_hbm.at[c_ax, sl])

    return k(rows, idx3d, zeros_nh)


def _sc_degree_count(idx3d, ones_rows, zeros_nh, H):
    _, NCH, CH = idx3d.shape
    NPAD = zeros_nh.shape[0]
    n_per_s = NPAD // _NS

    @functools.partial(
        pl.kernel,
        out_type=jax.ShapeDtypeStruct((_NC, NPAD, H), jnp.float32),
        mesh=_mesh(),
        compiler_params=_sc_params,
        scratch_types=[
            pltpu.VMEM((NCH, CH), jnp.int32),
            pltpu.VMEM((CH, H), jnp.float32),
            pltpu.VMEM_SHARED((NPAD, H), jnp.float32),
            pltpu.SemaphoreType.DMA,
        ],
    )
    def k(idx_hbm, ones_hbm, zeros_hbm, out_hbm, idx_v, ones_v, acc, sem):
        c_ax = lax.axis_index("c")
        s_ax = lax.axis_index("s")
        wid = s_ax * _NC + c_ax
        pltpu.sync_copy(zeros_hbm.at[pl.ds(s_ax * n_per_s, n_per_s)],
                        acc.at[pl.ds(s_ax * n_per_s, n_per_s)])
        pltpu.sync_copy(idx_hbm.at[wid], idx_v)
        pltpu.sync_copy(ones_hbm, ones_v)
        plsc.subcore_barrier()

        def issue(j, carry):
            pltpu.async_copy(ones_v, acc.at[idx_v.at[j]], sem, add=True)
            return carry

        lax.fori_loop(0, NCH, issue, 0)

        def drain(j, carry):
            pltpu.make_async_copy(ones_hbm, ones_v, sem).wait()
            return carry

        lax.fori_loop(0, NCH, drain, 0)
        plsc.subcore_barrier()
        sl = pl.ds(s_ax * n_per_s, n_per_s)
        pltpu.sync_copy(acc.at[sl], out_hbm.at[c_ax, sl])

    return k(idx3d, ones_rows, zeros_nh)


def _tc_input_proj(x, W0T, b0r):
    N = x.shape[0]
    H = W0T.shape[1]

    def body(x_ref, w_ref, b_ref, o_ref):
        o_ref[...] = jax.nn.relu(
            jnp.dot(x_ref[...], w_ref[...], preferred_element_type=jnp.float32)
            + b_ref[...])

    return pl.pallas_call(
        body, out_shape=jax.ShapeDtypeStruct((N, H), jnp.float32))(x, W0T, b0r)


def _tc_edge_mlp_t(ea_t, We1, be1c):
    K, E = We1.shape[0], ea_t.shape[1]
    EB = 16000
    grid = (E // EB,)

    def body(ea_ref, w_ref, b_ref, o_ref):
        f = jax.nn.relu(
            jnp.dot(w_ref[...], ea_ref[...], preferred_element_type=jnp.float32)
            + b_ref[...])
        o_ref[...] = f.astype(jnp.bfloat16)

    return pl.pallas_call(
        body,
        grid=grid,
        in_specs=[
            pl.BlockSpec((ea_t.shape[0], EB), lambda i: (0, i)),
            pl.BlockSpec(We1.shape, lambda i: (0, 0)),
            pl.BlockSpec(be1c.shape, lambda i: (0, 0)),
        ],
        out_specs=pl.BlockSpec((K, EB), lambda i: (0, i)),
        out_shape=jax.ShapeDtypeStruct((K, E), jnp.bfloat16),
    )(ea_t, We1, be1c)


def _tc_messages(f_t, xj, We2b, be2mT, H):
    K, E = f_t.shape
    EB = 3200
    NHALF = 5
    HB = EB // NHALF
    grid = (E // EB,)

    def body(ft_ref, xj_ref, w_ref, bm_ref, o_ref):
        xjT = xj_ref[...][:, 0:H].T
        bias = jnp.dot(bm_ref[...], xjT, preferred_element_type=jnp.float32)
        halves = []
        for h in range(NHALF):
            sl = slice(h * HB, (h + 1) * HB)
            wet = jnp.dot(w_ref[...], ft_ref[:, sl],
                          preferred_element_type=jnp.float32)
            acc = bias[:, sl]
            for i in range(H):
                acc = acc + xjT[i:i + 1, sl] * wet[i * H:(i + 1) * H, :]
            halves.append(acc)
        msg = jnp.concatenate(halves, axis=1).T
        o_ref[...] = jnp.concatenate(
            [msg, jnp.zeros((EB, 128 - H), jnp.float32)], axis=1)

    return pl.pallas_call(
        body,
        grid=grid,
        in_specs=[
            pl.BlockSpec((K, EB), lambda i: (0, i)),
            pl.BlockSpec((EB, 128), lambda i: (i, 0)),
            pl.BlockSpec(We2b.shape, lambda i: (0, 0)),
            pl.BlockSpec(be2mT.shape, lambda i: (0, 0)),
        ],
        out_specs=pl.BlockSpec((EB, 128), lambda i: (i, 0)),
        out_shape=jax.ShapeDtypeStruct((E, 128), jnp.float32),
    )(f_t, xj, We2b, be2mT)


def _tc_gru(parts, cparts, prev, rootm, biasr, gWihT, gWhhT, gbihr, gbhhr, H):
    N = prev.shape[0]
    NB = 1000
    grid = (N // NB,)

    def body(p_ref, c_ref, h_ref, root_ref, bias_ref, wih_ref, whh_ref,
             bih_ref, bhh_ref, o_ref):
        cnt = jnp.clip(c_ref[0, :, 0:1] + c_ref[1, :, 0:1], 1.0, None)
        agg = (p_ref[0] + p_ref[1]) / cnt
        h = h_ref[...]
        m = jax.nn.relu(
            agg + jnp.dot(h, root_ref[...], preferred_element_type=jnp.float32)
            + bias_ref[...])
        gi = jnp.dot(m, wih_ref[...], preferred_element_type=jnp.float32) + bih_ref[...]
        gh = jnp.dot(h, whh_ref[...], preferred_element_type=jnp.float32) + bhh_ref[...]
        r = jax.nn.sigmoid(gi[:, 0:H] + gh[:, 0:H])
        z = jax.nn.sigmoid(gi[:, H:2 * H] + gh[:, H:2 * H])
        n = jnp.tanh(gi[:, 2 * H:3 * H] + r * gh[:, 2 * H:3 * H])
        o_ref[...] = (1.0 - z) * n + z * h

    return pl.pallas_call(
        body,
        grid=grid,
        in_specs=[
            pl.BlockSpec((2, NB, H), lambda i: (0, i, 0)),
            pl.BlockSpec((2, NB, H), lambda i: (0, i, 0)),
            pl.BlockSpec((NB, H), lambda i: (i, 0)),
            pl.BlockSpec(rootm.shape, lambda i: (0, 0)),
            pl.BlockSpec(biasr.shape, lambda i: (0, 0)),
            pl.BlockSpec(gWihT.shape, lambda i: (0, 0)),
            pl.BlockSpec(gWhhT.shape, lambda i: (0, 0)),
            pl.BlockSpec(gbihr.shape, lambda i: (0, 0)),
            pl.BlockSpec(gbhhr.shape, lambda i: (0, 0)),
        ],
        out_specs=pl.BlockSpec((NB, H), lambda i: (i, 0)),
        out_shape=jax.ShapeDtypeStruct((N, H), jnp.float32),
    )(parts, cparts, prev, rootm, biasr, gWihT, gWhhT, gbihr, gbhhr)


def _tc_set2set(out, batch_col, sWihT, sWhhT, sbihr, sbhhr, W1T, b1r, W2T, b2r,
                B, H, PS):
    def body(out_ref, b_ref, wih_ref, whh_ref, bih_ref, bhh_ref,
             w1_ref, b1_ref, w2_ref, b2_ref, o_ref):
        nodes = out_ref[...]
        bcol = b_ref[...]
        onehot = (bcol == lax.broadcasted_iota(jnp.int32, (1, B), 1)
                  ).astype(jnp.float32)
        q_star = jnp.zeros((B, 2 * H), jnp.float32)
        hl = jnp.zeros((B, H), jnp.float32)
        cl = jnp.zeros((B, H), jnp.float32)
        for _ in range(PS):
            g = (jnp.dot(q_star, wih_ref[...], preferred_element_type=jnp.float32)
                 + bih_ref[...]
                 + jnp.dot(hl, whh_ref[...], preferred_element_type=jnp.float32)
                 + bhh_ref[...])
            cl = (jax.nn.sigmoid(g[:, H:2 * H]) * cl
                  + jax.nn.sigmoid(g[:, 0:H]) * jnp.tanh(g[:, 2 * H:3 * H]))
            hl = jax.nn.sigmoid(g[:, 3 * H:4 * H]) * jnp.tanh(cl)
            q_b = jnp.dot(onehot, hl, preferred_element_type=jnp.float32)
            e = jnp.sum(nodes * q_b, axis=1, keepdims=True)
            emax = jnp.max(jnp.where(onehot > 0.0, e, -jnp.inf), axis=0,
                           keepdims=True)
            emax_b = jnp.sum(onehot * emax, axis=1, keepdims=True)
            a = jnp.exp(e - emax_b)
            asum = jnp.sum(onehot * a, axis=0, keepdims=True)
            asum_b = jnp.sum(onehot * asum, axis=1, keepdims=True)
            a = a / (asum_b + 1e-16)
            r_ = lax.dot_general(onehot * a, nodes, (((0,), (0,)), ((), ())),
                                 preferred_element_type=jnp.float32)
            q_star = jnp.concatenate([hl, r_], axis=1)
        o = jax.nn.relu(
            jnp.dot(q_star, w1_ref[...], preferred_element_type=jnp.float32)
            + b1_ref[...])
        o_ref[...] = (jnp.dot(o, w2_ref[...], preferred_element_type=jnp.float32)
                      + b2_ref[...])

    return pl.pallas_call(
        body, out_shape=jax.ShapeDtypeStruct((B, 1), jnp.float32),
    )(out, batch_col, sWihT, sWhhT, sbihr, sbhhr, W1T, b1r, W2T, b2r)


def kernel(x, edge_index, edge_attr, batch, W0, b0, We1, be1, We2, be2, root,
           bias, gWih, gWhh, gbih, gbhh, sWih, sWhh, sbih, sbhh, W1, b1, W2,
           b2):
    N = x.shape[0]
    E = edge_index.shape[1]
    H = root.shape[0]
    B = 64
    L, PS = 3, 3
    NCH = E // (_NW * _CH)
    NPAD = ((N + 8 * _NS - 1) // (8 * _NS)) * (8 * _NS)

    src1d = edge_index[0]
    dst3d = edge_index[1].reshape(_NW, NCH, _CH)
    zeros_nh = jnp.zeros((NPAD, H), jnp.float32)
    ones_rows = jnp.ones((_CH, H), jnp.float32)
    batch_col = batch.reshape(N, 1)

    out = _tc_input_proj(x, W0.T, b0.reshape(1, H))
    f_t = _tc_edge_mlp_t(edge_attr.T, We1, be1.reshape(-1, 1))

    We2b = We2.astype(jnp.bfloat16)
    be2mT = be2.reshape(H, H).T
    gWihT, gWhhT = gWih.T, gWhh.T
    gbihr, gbhhr = gbih.reshape(1, -1), gbhh.reshape(1, -1)
    biasr = bias.reshape(1, H)

    cparts = _sc_degree_count(dst3d, ones_rows, zeros_nh, H)
    for layer in range(L):
        xj = _sc_gather(out, src1d, H)
        msg = _tc_messages(f_t, xj, We2b, be2mT, H)
        parts = _sc_scatter_add(msg, dst3d, zeros_nh, H)
        out = _tc_gru(parts, cparts, out, root, biasr, gWihT, gWhhT,
                      gbihr, gbhhr, H)

    o = _tc_set2set(out, batch_col, sWih.T, sWhh.T, sbih.reshape(1, -1),
                    sbhh.reshape(1, -1), W1.T, b1.reshape(1, H), W2.T,
                    b2.reshape(1, 1), B, H, PS)
    return o.reshape(-1)


# speedup vs baseline: 7.3055x; 1.0333x over previous
import functools

import jax
import jax.numpy as jnp
from jax import lax
from jax.experimental import pallas as pl
from jax.experimental.pallas import tpu as pltpu
from jax.experimental.pallas import tpu_sc as plsc

_NC, _NS = 2, 16
_NW = _NC * _NS
_CH = 40
_SEG = 25
_NSEG = 5
_SEG_E = _SEG * _CH

_sc_params = pltpu.CompilerParams(use_tc_tiling_on_sc=False)


def _mesh():
    return plsc.VectorSubcoreMesh(core_axis_name="c", subcore_axis_name="s")


def _sc_gather(nodes, idx1d, H):
    E = idx1d.shape[0]
    per_w = E // _NW

    @functools.partial(
        pl.kernel,
        out_type=jax.ShapeDtypeStruct((E, 128), jnp.float32),
        mesh=_mesh(),
        compiler_params=_sc_params,
        scratch_types=[
            pltpu.VMEM((per_w,), jnp.int32),
            pltpu.VMEM((_SEG_E, H), jnp.float32),
            pltpu.VMEM((_SEG_E, H), jnp.float32),
            pltpu.VMEM((_SEG_E, H), jnp.float32),
            pltpu.SemaphoreType.DMA,
            pltpu.SemaphoreType.DMA,
            pltpu.SemaphoreType.DMA,
            pltpu.SemaphoreType.DMA,
            pltpu.SemaphoreType.DMA,
            pltpu.SemaphoreType.DMA,
        ],
    )
    def k(nodes_hbm, idx_hbm, out_hbm, idx_v, b0, b1, b2,
          g0, g1, g2, w0, w1, w2):
        bufs, gs, ws = [b0, b1, b2], [g0, g1, g2], [w0, w1, w2]
        wid = lax.axis_index("s") * _NC + lax.axis_index("c")
        base = wid * per_w
        pltpu.sync_copy(idx_hbm.at[pl.ds(base, per_w)], idx_v)
        for s in range(_NSEG):
            b = s % 3
            if s >= 3:
                pltpu.make_async_copy(out_hbm.at[pl.ds(0, _SEG_E),
                                                 pl.ds(0, H)],
                                      bufs[b], ws[b]).wait()

            def issue(c, carry, _s=s, _b=b):
                off = (_s * _SEG + c) * _CH
                pltpu.async_copy(
                    nodes_hbm.at[idx_v.at[pl.ds(off, _CH)]],
                    bufs[_b].at[pl.ds(c * _CH, _CH)], gs[_b])
                return carry

            lax.fori_loop(0, _SEG, issue, 0)
            pltpu.make_async_copy(nodes_hbm.at[pl.ds(0, _SEG_E)],
                                  bufs[b], gs[b]).wait()
            pltpu.async_copy(bufs[b],
                             out_hbm.at[pl.ds(base + s * _SEG_E, _SEG_E),
                                        pl.ds(0, H)],
                             ws[b])
        for b in range(3):
            pltpu.make_async_copy(out_hbm.at[pl.ds(0, _SEG_E), pl.ds(0, H)],
                                  bufs[b], ws[b]).wait()

    return k(nodes, idx1d)


def _sc_scatter_add(rows, idx3d, zeros_nh, H):
    _, NCH, CH = idx3d.shape
    per_w = NCH * CH
    NPAD = zeros_nh.shape[0]
    n_per_s = NPAD // _NS

    @functools.partial(
        pl.kernel,
        out_type=jax.ShapeDtypeStruct((_NC, NPAD, H), jnp.float32),
        mesh=_mesh(),
        compiler_params=_sc_params,
        scratch_types=[
            pltpu.VMEM((NCH, CH), jnp.int32),
            pltpu.VMEM((_SEG_E, H), jnp.float32),
            pltpu.VMEM((_SEG_E, H), jnp.float32),
            pltpu.VMEM((_SEG_E, H), jnp.float32),
            pltpu.VMEM_SHARED((NPAD, H), jnp.float32),
            pltpu.SemaphoreType.DMA,
            pltpu.SemaphoreType.DMA,
            pltpu.SemaphoreType.DMA,
            pltpu.SemaphoreType.DMA,
            pltpu.SemaphoreType.DMA,
            pltpu.SemaphoreType.DMA,
        ],
    )
    def k(rows_hbm, idx_hbm, zeros_hbm, out_hbm, idx_v, b0, b1, b2, acc,
          r0, r1, r2, s0, s1, s2):
        bufs, rs, ss = [b0, b1, b2], [r0, r1, r2], [s0, s1, s2]
        c_ax = lax.axis_index("c")
        s_ax = lax.axis_index("s")
        wid = s_ax * _NC + c_ax
        base = wid * per_w
        pltpu.sync_copy(zeros_hbm.at[pl.ds(s_ax * n_per_s, n_per_s)],
                        acc.at[pl.ds(s_ax * n_per_s, n_per_s)])
        pltpu.sync_copy(idx_hbm.at[wid], idx_v)
        plsc.subcore_barrier()

        for s in range(3):
            pltpu.async_copy(
                rows_hbm.at[pl.ds(base + s * _SEG_E, _SEG_E), pl.ds(0, H)],
                bufs[s], rs[s])
        for s in range(_NSEG):
            b = s % 3
            pltpu.make_async_copy(rows_hbm.at[pl.ds(0, _SEG_E), pl.ds(0, H)],
                                  bufs[b], rs[b]).wait()

            def issue(c, carry, _s=s, _b=b):
                j = _s * _SEG + c
                pltpu.async_copy(bufs[_b].at[pl.ds(c * _CH, _CH)],
                                 acc.at[idx_v.at[j]], ss[_b], add=True)
                return carry

            lax.fori_loop(0, _SEG, issue, 0)
            if s + 3 < _NSEG:
                pltpu.make_async_copy(rows_hbm.at[pl.ds(0, _SEG_E),
                                                  pl.ds(0, H)],
                                      bufs[b], ss[b]).wait()
                pltpu.async_copy(
                    rows_hbm.at[pl.ds(base + (s + 3) * _SEG_E, _SEG_E),
                                pl.ds(0, H)],
                    bufs[b], rs[b])
        for b in range(3):
            pltpu.make_async_copy(rows_hbm.at[pl.ds(0, _SEG_E), pl.ds(0, H)],
                                  bufs[b], ss[b]).wait()
        plsc.subcore_barrier()
        sl = pl.ds(s_ax * n_per_s, n_per_s)
        pltpu.sync_copy(acc.at[sl], out_hbm.at[c_ax, sl])

    return k(rows, idx3d, zeros_nh)


def _sc_degree_count(idx3d, ones_rows, zeros_nh, H):
    _, NCH, CH = idx3d.shape
    NPAD = zeros_nh.shape[0]
    n_per_s = NPAD // _NS

    @functools.partial(
        pl.kernel,
        out_type=jax.ShapeDtypeStruct((_NC, NPAD, H), jnp.float32),
        mesh=_mesh(),
        compiler_params=_sc_params,
        scratch_types=[
            pltpu.VMEM((NCH, CH), jnp.int32),
            pltpu.VMEM((CH, H), jnp.float32),
            pltpu.VMEM_SHARED((NPAD, H), jnp.float32),
            pltpu.SemaphoreType.DMA,
        ],
    )
    def k(idx_hbm, ones_hbm, zeros_hbm, out_hbm, idx_v, ones_v, acc, sem):
        c_ax = lax.axis_index("c")
        s_ax = lax.axis_index("s")
        wid = s_ax * _NC + c_ax
        pltpu.sync_copy(zeros_hbm.at[pl.ds(s_ax * n_per_s, n_per_s)],
                        acc.at[pl.ds(s_ax * n_per_s, n_per_s)])
        pltpu.sync_copy(idx_hbm.at[wid], idx_v)
        pltpu.sync_copy(ones_hbm, ones_v)
        plsc.subcore_barrier()

        def issue(j, carry):
            pltpu.async_copy(ones_v, acc.at[idx_v.at[j]], sem, add=True)
            return carry

        lax.fori_loop(0, NCH, issue, 0)

        def drain(j, carry):
            pltpu.make_async_copy(ones_hbm, ones_v, sem).wait()
            return carry

        lax.fori_loop(0, NCH, drain, 0)
        plsc.subcore_barrier()
        sl = pl.ds(s_ax * n_per_s, n_per_s)
        pltpu.sync_copy(acc.at[sl], out_hbm.at[c_ax, sl])

    return k(idx3d, ones_rows, zeros_nh)


def _tc_input_proj(x, W0T, b0r):
    N = x.shape[0]
    H = W0T.shape[1]

    def body(x_ref, w_ref, b_ref, o_ref):
        o_ref[...] = jax.nn.relu(
            jnp.dot(x_ref[...], w_ref[...], preferred_element_type=jnp.float32)
            + b_ref[...])

    return pl.pallas_call(
        body, out_shape=jax.ShapeDtypeStruct((N, H), jnp.float32))(x, W0T, b0r)


def _tc_edge_mlp_t(ea_t, We1, be1c):
    K, E = We1.shape[0], ea_t.shape[1]
    EB = 16000
    grid = (E // EB,)

    def body(ea_ref, w_ref, b_ref, o_ref):
        f = jax.nn.relu(
            jnp.dot(w_ref[...], ea_ref[...], preferred_element_type=jnp.float32)
            + b_ref[...])
        o_ref[...] = f.astype(jnp.bfloat16)

    return pl.pallas_call(
        body,
        grid=grid,
        in_specs=[
            pl.BlockSpec((ea_t.shape[0], EB), lambda i: (0, i)),
            pl.BlockSpec(We1.shape, lambda i: (0, 0)),
            pl.BlockSpec(be1c.shape, lambda i: (0, 0)),
        ],
        out_specs=pl.BlockSpec((K, EB), lambda i: (0, i)),
        out_shape=jax.ShapeDtypeStruct((K, E), jnp.bfloat16),
    )(ea_t, We1, be1c)


def _tc_messages(f_t, xj, We2b, be2mT, H):
    K, E = f_t.shape
    EB = 6400
    NHALF = 10
    HB = EB // NHALF
    grid = (E // EB,)

    def body(ft_ref, xj_ref, w_ref, bm_ref, o_ref):
        xjT = xj_ref[...][:, 0:H].T
        bias = jnp.dot(bm_ref[...], xjT, preferred_element_type=jnp.float32)
        halves = []
        for h in range(NHALF):
            sl = slice(h * HB, (h + 1) * HB)
            wet = jnp.dot(w_ref[...], ft_ref[:, sl],
                          preferred_element_type=jnp.float32)
            acc = bias[:, sl]
            for i in range(H):
                acc = acc + xjT[i:i + 1, sl] * wet[i * H:(i + 1) * H, :]
            halves.append(acc)
        msg = jnp.concatenate(halves, axis=1).T
        o_ref[...] = jnp.concatenate(
            [msg, jnp.zeros((EB, 128 - H), jnp.float32)], axis=1)

    return pl.pallas_call(
        body,
        grid=grid,
        in_specs=[
            pl.BlockSpec((K, EB), lambda i: (0, i)),
            pl.BlockSpec((EB, 128), lambda i: (i, 0)),
            pl.BlockSpec(We2b.shape, lambda i: (0, 0)),
            pl.BlockSpec(be2mT.shape, lambda i: (0, 0)),
        ],
        out_specs=pl.BlockSpec((EB, 128), lambda i: (i, 0)),
        out_shape=jax.ShapeDtypeStruct((E, 128), jnp.float32),
    )(f_t, xj, We2b, be2mT)


def _tc_gru(parts, cparts, prev, rootm, biasr, gWihT, gWhhT, gbihr, gbhhr, H):
    N = prev.shape[0]
    NB = 2000
    grid = (N // NB,)

    def body(p_ref, c_ref, h_ref, root_ref, bias_ref, wih_ref, whh_ref,
             bih_ref, bhh_ref, o_ref):
        cnt = jnp.clip(c_ref[0, :, 0:1] + c_ref[1, :, 0:1], 1.0, None)
        agg = (p_ref[0] + p_ref[1]) / cnt
        h = h_ref[...]
        m = jax.nn.relu(
            agg + jnp.dot(h, root_ref[...], preferred_element_type=jnp.float32)
            + bias_ref[...])
        gi = jnp.dot(m, wih_ref[...], preferred_element_type=jnp.float32) + bih_ref[...]
        gh = jnp.dot(h, whh_ref[...], preferred_element_type=jnp.float32) + bhh_ref[...]
        r = jax.nn.sigmoid(gi[:, 0:H] + gh[:, 0:H])
        z = jax.nn.sigmoid(gi[:, H:2 * H] + gh[:, H:2 * H])
        n = jnp.tanh(gi[:, 2 * H:3 * H] + r * gh[:, 2 * H:3 * H])
        o_ref[...] = (1.0 - z) * n + z * h

    return pl.pallas_call(
        body,
        grid=grid,
        in_specs=[
            pl.BlockSpec((2, NB, H), lambda i: (0, i, 0)),
            pl.BlockSpec((2, NB, H), lambda i: (0, i, 0)),
            pl.BlockSpec((NB, H), lambda i: (i, 0)),
            pl.BlockSpec(rootm.shape, lambda i: (0, 0)),
            pl.BlockSpec(biasr.shape, lambda i: (0, 0)),
            pl.BlockSpec(gWihT.shape, lambda i: (0, 0)),
            pl.BlockSpec(gWhhT.shape, lambda i: (0, 0)),
            pl.BlockSpec(gbihr.shape, lambda i: (0, 0)),
            pl.BlockSpec(gbhhr.shape, lambda i: (0, 0)),
        ],
        out_specs=pl.BlockSpec((NB, H), lambda i: (i, 0)),
        out_shape=jax.ShapeDtypeStruct((N, H), jnp.float32),
    )(parts, cparts, prev, rootm, biasr, gWihT, gWhhT, gbihr, gbhhr)


def _tc_set2set(out, batch_col, sWihT, sWhhT, sbihr, sbhhr, W1T, b1r, W2T, b2r,
                B, H, PS):
    def body(out_ref, b_ref, wih_ref, whh_ref, bih_ref, bhh_ref,
             w1_ref, b1_ref, w2_ref, b2_ref, o_ref):
        nodes = out_ref[...]
        bcol = b_ref[...]
        onehot = (bcol == lax.broadcasted_iota(jnp.int32, (1, B), 1)
                  ).astype(jnp.float32)
        q_star = jnp.zeros((B, 2 * H), jnp.float32)
        hl = jnp.zeros((B, H), jnp.float32)
        cl = jnp.zeros((B, H), jnp.float32)
        for _ in range(PS):
            g = (jnp.dot(q_star, wih_ref[...], preferred_element_type=jnp.float32)
                 + bih_ref[...]
                 + jnp.dot(hl, whh_ref[...], preferred_element_type=jnp.float32)
                 + bhh_ref[...])
            cl = (jax.nn.sigmoid(g[:, H:2 * H]) * cl
                  + jax.nn.sigmoid(g[:, 0:H]) * jnp.tanh(g[:, 2 * H:3 * H]))
            hl = jax.nn.sigmoid(g[:, 3 * H:4 * H]) * jnp.tanh(cl)
            q_b = jnp.dot(onehot, hl, preferred_element_type=jnp.float32)
            e = jnp.sum(nodes * q_b, axis=1, keepdims=True)
            emax = jnp.max(jnp.where(onehot > 0.0, e, -jnp.inf), axis=0,
                           keepdims=True)
            emax_b = jnp.sum(onehot * emax, axis=1, keepdims=True)
            a = jnp.exp(e - emax_b)
            asum = jnp.sum(onehot * a, axis=0, keepdims=True)
            asum_b = jnp.sum(onehot * asum, axis=1, keepdims=True)
            a = a / (asum_b + 1e-16)
            r_ = lax.dot_general(onehot * a, nodes, (((0,), (0,)), ((), ())),
                                 preferred_element_type=jnp.float32)
            q_star = jnp.concatenate([hl, r_], axis=1)
        o = jax.nn.relu(
            jnp.dot(q_star, w1_ref[...], preferred_element_type=jnp.float32)
            + b1_ref[...])
        o_ref[...] = (jnp.dot(o, w2_ref[...], preferred_element_type=jnp.float32)
                      + b2_ref[...])

    return pl.pallas_call(
        body, out_shape=jax.ShapeDtypeStruct((B, 1), jnp.float32),
    )(out, batch_col, sWihT, sWhhT, sbihr, sbhhr, W1T, b1r, W2T, b2r)


def kernel(x, edge_index, edge_attr, batch, W0, b0, We1, be1, We2, be2, root,
           bias, gWih, gWhh, gbih, gbhh, sWih, sWhh, sbih, sbhh, W1, b1, W2,
           b2):
    N = x.shape[0]
    E = edge_index.shape[1]
    H = root.shape[0]
    B = 64
    L, PS = 3, 3
    NCH = E // (_NW * _CH)
    NPAD = ((N + 8 * _NS - 1) // (8 * _NS)) * (8 * _NS)

    src1d = edge_index[0]
    dst3d = edge_index[1].reshape(_NW, NCH, _CH)
    zeros_nh = jnp.zeros((NPAD, H), jnp.float32)
    ones_rows = jnp.ones((_CH, H), jnp.float32)
    batch_col = batch.reshape(N, 1)

    out = _tc_input_proj(x, W0.T, b0.reshape(1, H))
    f_t = _tc_edge_mlp_t(edge_attr.T, We1, be1.reshape(-1, 1))

    We2b = We2.astype(jnp.bfloat16)
    be2mT = be2.reshape(H, H).T
    gWihT, gWhhT = gWih.T, gWhh.T
    gbihr, gbhhr = gbih.reshape(1, -1), gbhh.reshape(1, -1)
    biasr = bias.reshape(1, H)

    cparts = _sc_degree_count(dst3d, ones_rows, zeros_nh, H)
    for layer in range(L):
        xj = _sc_gather(out, src1d, H)
        msg = _tc_messages(f_t, xj, We2b, be2mT, H)
        parts = _sc_scatter_add(msg, dst3d, zeros_nh, H)
        out = _tc_gru(parts, cparts, out, root, biasr, gWihT, gWhhT,
                      gbihr, gbhhr, H)

    o = _tc_set2set(out, batch_col, sWih.T, sWhh.T, sbih.reshape(1, -1),
                    sbhh.reshape(1, -1), W1.T, b1.reshape(1, H), W2.T,
                    b2.reshape(1, 1), B, H, PS)
    return o.reshape(-1)


# speedup vs baseline: 7.3218x; 1.0022x over previous
import functools

import jax
import jax.numpy as jnp
from jax import lax
from jax.experimental import pallas as pl
from jax.experimental.pallas import tpu as pltpu
from jax.experimental.pallas import tpu_sc as plsc

_NC, _NS = 2, 16
_NW = _NC * _NS
_CH = 40
_SEG = 25
_NSEG = 5
_SEG_E = _SEG * _CH

_sc_params = pltpu.CompilerParams(use_tc_tiling_on_sc=False)


def _mesh():
    return plsc.VectorSubcoreMesh(core_axis_name="c", subcore_axis_name="s")


def _sc_gather(nodes, idx1d, H):
    E = idx1d.shape[0]
    per_w = E // _NW

    @functools.partial(
        pl.kernel,
        out_type=jax.ShapeDtypeStruct((E, 128), jnp.float32),
        mesh=_mesh(),
        compiler_params=_sc_params,
        scratch_types=[
            pltpu.VMEM((per_w,), jnp.int32),
            pltpu.VMEM((_SEG_E, H), jnp.float32),
            pltpu.VMEM((_SEG_E, H), jnp.float32),
            pltpu.VMEM((_SEG_E, H), jnp.float32),
            pltpu.SemaphoreType.DMA,
            pltpu.SemaphoreType.DMA,
            pltpu.SemaphoreType.DMA,
            pltpu.SemaphoreType.DMA,
            pltpu.SemaphoreType.DMA,
            pltpu.SemaphoreType.DMA,
        ],
    )
    def k(nodes_hbm, idx_hbm, out_hbm, idx_v, b0, b1, b2,
          g0, g1, g2, w0, w1, w2):
        bufs, gs, ws = [b0, b1, b2], [g0, g1, g2], [w0, w1, w2]
        wid = lax.axis_index("s") * _NC + lax.axis_index("c")
        base = wid * per_w
        pltpu.sync_copy(idx_hbm.at[pl.ds(base, per_w)], idx_v)
        for s in range(_NSEG):
            b = s % 3
            if s >= 3:
                pltpu.make_async_copy(out_hbm.at[pl.ds(0, _SEG_E),
                                                 pl.ds(0, H)],
                                      bufs[b], ws[b]).wait()

            def issue(c, carry, _s=s, _b=b):
                off = (_s * _SEG + c) * _CH
                pltpu.async_copy(
                    nodes_hbm.at[idx_v.at[pl.ds(off, _CH)]],
                    bufs[_b].at[pl.ds(c * _CH, _CH)], gs[_b])
                return carry

            lax.fori_loop(0, _SEG, issue, 0)
            pltpu.make_async_copy(nodes_hbm.at[pl.ds(0, _SEG_E)],
                                  bufs[b], gs[b]).wait()
            pltpu.async_copy(bufs[b],
                             out_hbm.at[pl.ds(base + s * _SEG_E, _SEG_E),
                                        pl.ds(0, H)],
                             ws[b])
        for b in range(3):
            pltpu.make_async_copy(out_hbm.at[pl.ds(0, _SEG_E), pl.ds(0, H)],
                                  bufs[b], ws[b]).wait()

    return k(nodes, idx1d)


def _sc_scatter_add(rows, idx3d, zeros_nh, H):
    _, NCH, CH = idx3d.shape
    per_w = NCH * CH
    NPAD = zeros_nh.shape[0]
    n_per_s = NPAD // _NS

    @functools.partial(
        pl.kernel,
        out_type=jax.ShapeDtypeStruct((_NC, NPAD, H), jnp.float32),
        mesh=_mesh(),
        compiler_params=_sc_params,
        scratch_types=[
            pltpu.VMEM((NCH, CH), jnp.int32),
            pltpu.VMEM((_SEG_E, H), jnp.float32),
            pltpu.VMEM((_SEG_E, H), jnp.float32),
            pltpu.VMEM((_SEG_E, H), jnp.float32),
            pltpu.VMEM_SHARED((NPAD, H), jnp.float32),
            pltpu.SemaphoreType.DMA,
            pltpu.SemaphoreType.DMA,
            pltpu.SemaphoreType.DMA,
            pltpu.SemaphoreType.DMA,
            pltpu.SemaphoreType.DMA,
            pltpu.SemaphoreType.DMA,
        ],
    )
    def k(rows_hbm, idx_hbm, zeros_hbm, out_hbm, idx_v, b0, b1, b2, acc,
          r0, r1, r2, s0, s1, s2):
        bufs, rs, ss = [b0, b1, b2], [r0, r1, r2], [s0, s1, s2]
        c_ax = lax.axis_index("c")
        s_ax = lax.axis_index("s")
        wid = s_ax * _NC + c_ax
        base = wid * per_w
        pltpu.sync_copy(zeros_hbm.at[pl.ds(s_ax * n_per_s, n_per_s)],
                        acc.at[pl.ds(s_ax * n_per_s, n_per_s)])
        pltpu.sync_copy(idx_hbm.at[wid], idx_v)
        plsc.subcore_barrier()

        for s in range(3):
            pltpu.async_copy(
                rows_hbm.at[pl.ds(base + s * _SEG_E, _SEG_E), pl.ds(0, H)],
                bufs[s], rs[s])
        for s in range(_NSEG):
            b = s % 3
            pltpu.make_async_copy(rows_hbm.at[pl.ds(0, _SEG_E), pl.ds(0, H)],
                                  bufs[b], rs[b]).wait()

            def issue(c, carry, _s=s, _b=b):
                j = _s * _SEG + c
                pltpu.async_copy(bufs[_b].at[pl.ds(c * _CH, _CH)],
                                 acc.at[idx_v.at[j]], ss[_b], add=True)
                return carry

            lax.fori_loop(0, _SEG, issue, 0)
            if s + 3 < _NSEG:
                pltpu.make_async_copy(rows_hbm.at[pl.ds(0, _SEG_E),
                                                  pl.ds(0, H)],
                                      bufs[b], ss[b]).wait()
                pltpu.async_copy(
                    rows_hbm.at[pl.ds(base + (s + 3) * _SEG_E, _SEG_E),
                                pl.ds(0, H)],
                    bufs[b], rs[b])
        for b in range(3):
            pltpu.make_async_copy(rows_hbm.at[pl.ds(0, _SEG_E), pl.ds(0, H)],
                                  bufs[b], ss[b]).wait()
        plsc.subcore_barrier()
        sl = pl.ds(s_ax * n_per_s, n_per_s)
        pltpu.sync_copy(acc.at[sl], out_hbm.at[c_ax, sl])

    return k(rows, idx3d, zeros_nh)


def _sc_degree_count(idx3d, ones_rows, zeros_nh, H):
    _, NCH, CH = idx3d.shape
    NPAD = zeros_nh.shape[0]
    n_per_s = NPAD // _NS

    @functools.partial(
        pl.kernel,
        out_type=jax.ShapeDtypeStruct((_NC, NPAD, H), jnp.float32),
        mesh=_mesh(),
        compiler_params=_sc_params,
        scratch_types=[
            pltpu.VMEM((NCH, CH), jnp.int32),
            pltpu.VMEM((CH, H), jnp.float32),
            pltpu.VMEM_SHARED((NPAD, H), jnp.float32),
            pltpu.SemaphoreType.DMA,
        ],
    )
    def k(idx_hbm, ones_hbm, zeros_hbm, out_hbm, idx_v, ones_v, acc, sem):
        c_ax = lax.axis_index("c")
        s_ax = lax.axis_index("s")
        wid = s_ax * _NC + c_ax
        pltpu.sync_copy(zeros_hbm.at[pl.ds(s_ax * n_per_s, n_per_s)],
                        acc.at[pl.ds(s_ax * n_per_s, n_per_s)])
        pltpu.sync_copy(idx_hbm.at[wid], idx_v)
        pltpu.sync_copy(ones_hbm, ones_v)
        plsc.subcore_barrier()

        def issue(j, carry):
            pltpu.async_copy(ones_v, acc.at[idx_v.at[j]], sem, add=True)
            return carry

        lax.fori_loop(0, NCH, issue, 0)

        def drain(j, carry):
            pltpu.make_async_copy(ones_hbm, ones_v, sem).wait()
            return carry

        lax.fori_loop(0, NCH, drain, 0)
        plsc.subcore_barrier()
        sl = pl.ds(s_ax * n_per_s, n_per_s)
        pltpu.sync_copy(acc.at[sl], out_hbm.at[c_ax, sl])

    return k(idx3d, ones_rows, zeros_nh)


def _tc_input_proj(x, W0T, b0r):
    N = x.shape[0]
    H = W0T.shape[1]

    def body(x_ref, w_ref, b_ref, o_ref):
        o_ref[...] = jax.nn.relu(
            jnp.dot(x_ref[...], w_ref[...], preferred_element_type=jnp.float32)
            + b_ref[...])

    return pl.pallas_call(
        body, out_shape=jax.ShapeDtypeStruct((N, H), jnp.float32))(x, W0T, b0r)


def _tc_edge_mlp_t(ea_t, We1, be1c):
    K, E = We1.shape[0], ea_t.shape[1]
    EB = 16000
    grid = (E // EB,)

    def body(ea_ref, w_ref, b_ref, o_ref):
        f = jax.nn.relu(
            jnp.dot(w_ref[...], ea_ref[...], preferred_element_type=jnp.float32)
            + b_ref[...])
        o_ref[...] = f.astype(jnp.bfloat16)

    return pl.pallas_call(
        body,
        grid=grid,
        in_specs=[
            pl.BlockSpec((ea_t.shape[0], EB), lambda i: (0, i)),
            pl.BlockSpec(We1.shape, lambda i: (0, 0)),
            pl.BlockSpec(be1c.shape, lambda i: (0, 0)),
        ],
        out_specs=pl.BlockSpec((K, EB), lambda i: (0, i)),
        out_shape=jax.ShapeDtypeStruct((K, E), jnp.bfloat16),
    )(ea_t, We1, be1c)


def _tc_messages(f_t, xj, We2b, be2mT, H):
    K, E = f_t.shape
    EB = 6400
    NHALF = 10
    HB = EB // NHALF
    grid = (E // EB,)

    def body(ft_ref, xj_ref, w_ref, bm_ref, o_ref):
        xjT = xj_ref[...][:, 0:H].T
        bias = jnp.dot(bm_ref[...], xjT, preferred_element_type=jnp.float32)
        halves = []
        for h in range(NHALF):
            sl = slice(h * HB, (h + 1) * HB)
            wet = jnp.dot(w_ref[...], ft_ref[:, sl],
                          preferred_element_type=jnp.float32)
            acc = bias[:, sl]
            for i in range(H):
                acc = acc + xjT[i:i + 1, sl] * wet[i * H:(i + 1) * H, :]
            halves.append(acc)
        o_ref[:, 0:H] = jnp.concatenate(halves, axis=1).T

    return pl.pallas_call(
        body,
        grid=grid,
        in_specs=[
            pl.BlockSpec((K, EB), lambda i: (0, i)),
            pl.BlockSpec((EB, 128), lambda i: (i, 0)),
            pl.BlockSpec(We2b.shape, lambda i: (0, 0)),
            pl.BlockSpec(be2mT.shape, lambda i: (0, 0)),
        ],
        out_specs=pl.BlockSpec((EB, 128), lambda i: (i, 0)),
        out_shape=jax.ShapeDtypeStruct((E, 128), jnp.float32),
    )(f_t, xj, We2b, be2mT)


def _tc_gru(parts, cparts, prev, rootm, biasr, gWihT, gWhhT, gbihr, gbhhr, H):
    N = prev.shape[0]
    NB = 2000
    grid = (N // NB,)

    def body(p_ref, c_ref, h_ref, root_ref, bias_ref, wih_ref, whh_ref,
             bih_ref, bhh_ref, o_ref):
        cnt = jnp.clip(c_ref[0, :, 0:1] + c_ref[1, :, 0:1], 1.0, None)
        agg = (p_ref[0] + p_ref[1]) / cnt
        h = h_ref[...]
        m = jax.nn.relu(
            agg + jnp.dot(h, root_ref[...], preferred_element_type=jnp.float32)
            + bias_ref[...])
        gi = jnp.dot(m, wih_ref[...], preferred_element_type=jnp.float32) + bih_ref[...]
        gh = jnp.dot(h, whh_ref[...], preferred_element_type=jnp.float32) + bhh_ref[...]
        r = jax.nn.sigmoid(gi[:, 0:H] + gh[:, 0:H])
        z = jax.nn.sigmoid(gi[:, H:2 * H] + gh[:, H:2 * H])
        n = jnp.tanh(gi[:, 2 * H:3 * H] + r * gh[:, 2 * H:3 * H])
        o_ref[...] = (1.0 - z) * n + z * h

    return pl.pallas_call(
        body,
        grid=grid,
        in_specs=[
            pl.BlockSpec((2, NB, H), lambda i: (0, i, 0)),
            pl.BlockSpec((2, NB, H), lambda i: (0, i, 0)),
            pl.BlockSpec((NB, H), lambda i: (i, 0)),
            pl.BlockSpec(rootm.shape, lambda i: (0, 0)),
            pl.BlockSpec(biasr.shape, lambda i: (0, 0)),
            pl.BlockSpec(gWihT.shape, lambda i: (0, 0)),
            pl.BlockSpec(gWhhT.shape, lambda i: (0, 0)),
            pl.BlockSpec(gbihr.shape, lambda i: (0, 0)),
            pl.BlockSpec(gbhhr.shape, lambda i: (0, 0)),
        ],
        out_specs=pl.BlockSpec((NB, H), lambda i: (i, 0)),
        out_shape=jax.ShapeDtypeStruct((N, H), jnp.float32),
    )(parts, cparts, prev, rootm, biasr, gWihT, gWhhT, gbihr, gbhhr)


def _tc_set2set(out, batch_col, sWihT, sWhhT, sbihr, sbhhr, W1T, b1r, W2T, b2r,
                B, H, PS):
    def body(out_ref, b_ref, wih_ref, whh_ref, bih_ref, bhh_ref,
             w1_ref, b1_ref, w2_ref, b2_ref, o_ref):
        nodes = out_ref[...]
        bcol = b_ref[...]
        onehot = (bcol == lax.broadcasted_iota(jnp.int32, (1, B), 1)
                  ).astype(jnp.float32)
        q_star = jnp.zeros((B, 2 * H), jnp.float32)
        hl = jnp.zeros((B, H), jnp.float32)
        cl = jnp.zeros((B, H), jnp.float32)
        for _ in range(PS):
            g = (jnp.dot(q_star, wih_ref[...], preferred_element_type=jnp.float32)
                 + bih_ref[...]
                 + jnp.dot(hl, whh_ref[...], preferred_element_type=jnp.float32)
                 + bhh_ref[...])
            cl = (jax.nn.sigmoid(g[:, H:2 * H]) * cl
                  + jax.nn.sigmoid(g[:, 0:H]) * jnp.tanh(g[:, 2 * H:3 * H]))
            hl = jax.nn.sigmoid(g[:, 3 * H:4 * H]) * jnp.tanh(cl)
            q_b = jnp.dot(onehot, hl, preferred_element_type=jnp.float32)
            e = jnp.sum(nodes * q_b, axis=1, keepdims=True)
            emax = jnp.max(jnp.where(onehot > 0.0, e, -jnp.inf), axis=0,
                           keepdims=True)
            emax_b = jnp.sum(onehot * emax, axis=1, keepdims=True)
            a = jnp.exp(e - emax_b)
            asum = jnp.sum(onehot * a, axis=0, keepdims=True)
            asum_b = jnp.sum(onehot * asum, axis=1, keepdims=True)
            a = a / (asum_b + 1e-16)
            r_ = lax.dot_general(onehot * a, nodes, (((0,), (0,)), ((), ())),
                                 preferred_element_type=jnp.float32)
            q_star = jnp.concatenate([hl, r_], axis=1)
        o = jax.nn.relu(
            jnp.dot(q_star, w1_ref[...], preferred_element_type=jnp.float32)
            + b1_ref[...])
        o_ref[...] = (jnp.dot(o, w2_ref[...], preferred_element_type=jnp.float32)
                      + b2_ref[...])

    return pl.pallas_call(
        body, out_shape=jax.ShapeDtypeStruct((B, 1), jnp.float32),
    )(out, batch_col, sWihT, sWhhT, sbihr, sbhhr, W1T, b1r, W2T, b2r)


def kernel(x, edge_index, edge_attr, batch, W0, b0, We1, be1, We2, be2, root,
           bias, gWih, gWhh, gbih, gbhh, sWih, sWhh, sbih, sbhh, W1, b1, W2,
           b2):
    N = x.shape[0]
    E = edge_index.shape[1]
    H = root.shape[0]
    B = 64
    L, PS = 3, 3
    NCH = E // (_NW * _CH)
    NPAD = ((N + 8 * _NS - 1) // (8 * _NS)) * (8 * _NS)

    src1d = edge_index[0]
    dst3d = edge_index[1].reshape(_NW, NCH, _CH)
    zeros_nh = jnp.zeros((NPAD, H), jnp.float32)
    ones_rows = jnp.ones((_CH, H), jnp.float32)
    batch_col = batch.reshape(N, 1)

    out = _tc_input_proj(x, W0.T, b0.reshape(1, H))
    f_t = _tc_edge_mlp_t(edge_attr.T, We1, be1.reshape(-1, 1))

    We2b = We2.astype(jnp.bfloat16)
    be2mT = be2.reshape(H, H).T
    gWihT, gWhhT = gWih.T, gWhh.T
    gbihr, gbhhr = gbih.reshape(1, -1), gbhh.reshape(1, -1)
    biasr = bias.reshape(1, H)

    cparts = _sc_degree_count(dst3d, ones_rows, zeros_nh, H)
    for layer in range(L):
        xj = _sc_gather(out, src1d, H)
        msg = _tc_messages(f_t, xj, We2b, be2mT, H)
        parts = _sc_scatter_add(msg, dst3d, zeros_nh, H)
        out = _tc_gru(parts, cparts, out, root, biasr, gWihT, gWhhT,
                      gbihr, gbhhr, H)

    o = _tc_set2set(out, batch_col, sWih.T, sWhh.T, sbih.reshape(1, -1),
                    sbhh.reshape(1, -1), W1.T, b1.reshape(1, H), W2.T,
                    b2.reshape(1, 1), B, H, PS)
    return o.reshape(-1)
